# speedup vs baseline: 1.0008x; 1.0008x over previous
import functools

import jax
import jax.numpy as jnp
from jax.experimental import pallas as pl
from jax.experimental.pallas import tpu as pltpu


def _gat(x, src, dst, W, a_src, a_dst, b, H, C, n):
    h = (x @ W).reshape(n, H, C)
    alpha_s = (h * a_src[None]).sum(-1)
    alpha_d = (h * a_dst[None]).sum(-1)
    e = jax.nn.leaky_relu(alpha_s[src] + alpha_d[dst], 0.2)
    m = jax.ops.segment_max(e, dst, num_segments=n)
    ex = jnp.exp(e - m[dst])
    denom = jax.ops.segment_sum(ex, dst, num_segments=n)
    alpha = ex / (denom[dst] + 1e-16)
    out = jax.ops.segment_sum(h[src] * alpha[:, :, None], dst, num_segments=n)
    return out.reshape(n, H * C) + b


def _head_kernel(h_ref, batch_ref, fc1w_ref, fc1b_ref, fc3w_ref, fc3b_ref,
                 out_ref, sums_ref, cnt_ref, *, n_graphs, nb):
    i = pl.program_id(0)

    @pl.when(i == 0)
    def _():
        sums_ref[...] = jnp.zeros_like(sums_ref)
        cnt_ref[...] = jnp.zeros_like(cnt_ref)

    h = h_ref[...]
    ids = batch_ref[...]
    onehot = (ids == jax.lax.broadcasted_iota(jnp.int32, (1, n_graphs), 1)
              ).astype(jnp.float32)
    sums_ref[...] += jax.lax.dot_general(
        onehot, h, (((0,), (0,)), ((), ())),
        preferred_element_type=jnp.float32)
    cnt_ref[...] += jnp.sum(onehot, axis=0, keepdims=True)

    @pl.when(i == nb - 1)
    def _():
        g = sums_ref[...] / jnp.maximum(cnt_ref[...], 1.0).T
        g = jax.nn.relu(g @ fc1w_ref[...] + fc1b_ref[...])
        out_ref[...] = g @ fc3w_ref[...] + fc3b_ref[...]


def kernel(x, edge_index, batch, W1, att_src1, att_dst1, b1, W2, att_src2,
           att_dst2, b2, Wg_root, Wg_nbr, bg, Ws_nbr, Ws_root, bs, fc1_w,
           fc1_b, fc3_w, fc3_b):
    n = x.shape[0]
    H1, C1 = att_src1.shape
    H2, C2 = att_src2.shape
    n_graphs = 64
    src, dst = edge_index[0], edge_index[1]
    ar = jnp.arange(n, dtype=src.dtype)
    src_sl = jnp.concatenate([src, ar])
    dst_sl = jnp.concatenate([dst, ar])
    h = jax.nn.relu(_gat(x, src_sl, dst_sl, W1, att_src1, att_dst1, b1, H1, C1, n))
    h = jax.nn.relu(_gat(h, src_sl, dst_sl, W2, att_src2, att_dst2, b2, H2, C2, n))
    nbr = jax.ops.segment_sum(h[src], dst, num_segments=n)
    h = jax.nn.relu(h @ Wg_root + nbr @ Wg_nbr + bg)
    nbr_sum = jax.ops.segment_sum(h[src], dst, num_segments=n)
    deg = jax.ops.segment_sum(jnp.ones((src.shape[0],), dtype=h.dtype), dst,
                              num_segments=n)
    nbr_mean = nbr_sum / jnp.maximum(deg, 1.0)[:, None]
    h = jax.nn.relu(nbr_mean @ Ws_nbr + h @ Ws_root + bs)

    bn = 2000
    nb = n // bn
    d = h.shape[1]
    out = pl.pallas_call(
        functools.partial(_head_kernel, n_graphs=n_graphs, nb=nb),
        grid=(nb,),
        in_specs=[
            pl.BlockSpec((bn, d), lambda i: (i, 0)),
            pl.BlockSpec((bn, 1), lambda i: (i, 0)),
            pl.BlockSpec(fc1_w.shape, lambda i: (0, 0)),
            pl.BlockSpec((1, fc1_b.shape[0]), lambda i: (0, 0)),
            pl.BlockSpec(fc3_w.shape, lambda i: (0, 0)),
            pl.BlockSpec((1, fc3_b.shape[0]), lambda i: (0, 0)),
        ],
        out_specs=pl.BlockSpec((n_graphs, fc3_w.shape[1]), lambda i: (0, 0)),
        out_shape=jax.ShapeDtypeStruct((n_graphs, fc3_w.shape[1]), jnp.float32),
        scratch_shapes=[
            pltpu.VMEM((n_graphs, d), jnp.float32),
            pltpu.VMEM((1, n_graphs), jnp.float32),
        ],
    )(h, batch.reshape(n, 1), fc1_w, fc1_b.reshape(1, -1), fc3_w,
      fc3_b.reshape(1, -1))
    return out


# speedup vs baseline: 11.6487x; 11.6389x over previous
import functools

import jax
import jax.numpy as jnp
from jax import lax
from jax.experimental import pallas as pl
from jax.experimental.pallas import tpu as pltpu
from jax.experimental.pallas import tpu_sc as plsc

N = 10000
E = 320000
G = 64
H = 16
CP = 64
BN = 400
NB = N // BN
NP = 10240
NTILE = 16
CH = 1000
_mesh = plsc.VectorSubcoreMesh(core_axis_name="c", subcore_axis_name="s")
_sc_params = pltpu.CompilerParams(use_tc_tiling_on_sc=False, needs_layout_passes=False)


def _mm_body(x_ref, w_ref, ht_ref):
    ht_ref[0] = jnp.dot(x_ref[...], w_ref[0],
                        preferred_element_type=jnp.float32)


def _mm(x, wt):
    d = x.shape[1]
    return pl.pallas_call(
        _mm_body,
        grid=(NB, H),
        in_specs=[
            pl.BlockSpec((BN, d), lambda i, k: (i, 0)),
            pl.BlockSpec((1, d, CP), lambda i, k: (k, 0, 0)),
        ],
        out_specs=pl.BlockSpec((1, BN, CP), lambda i, k: (k, i, 0)),
        out_shape=jax.ShapeDtypeStruct((H, N, CP), jnp.float32),
    )(x, wt)


def _asad_body(ht_ref, asr_ref, adr_ref, as_ref, ad_ref):
    cols_s = []
    cols_d = []
    for k in range(H):
        h = ht_ref[k]
        cols_s.append(jnp.sum(h * asr_ref[k], axis=1, keepdims=True))
        cols_d.append(jnp.sum(h * adr_ref[k], axis=1, keepdims=True))
    as_ref[...] = jnp.concatenate(cols_s, axis=1)
    ad_ref[...] = jnp.concatenate(cols_d, axis=1)


def _asad(ht, asr, adr):
    return pl.pallas_call(
        _asad_body,
        grid=(NB,),
        in_specs=[
            pl.BlockSpec((H, BN, CP), lambda i: (0, i, 0)),
            pl.BlockSpec((H, 1, CP), lambda i: (0, 0, 0)),
            pl.BlockSpec((H, 1, CP), lambda i: (0, 0, 0)),
        ],
        out_specs=[
            pl.BlockSpec((BN, H), lambda i: (i, 0)),
            pl.BlockSpec((BN, H), lambda i: (i, 0)),
        ],
        out_shape=[
            jax.ShapeDtypeStruct((N, H), jnp.float32),
            jax.ShapeDtypeStruct((N, H), jnp.float32),
        ],
    )(ht, asr, adr)


def _prep_body(as_ref, ad_ref, adm_ref, exself_ref):
    a_s = as_ref[...]
    a_d = ad_ref[...]
    mb = jnp.max(a_s, axis=0, keepdims=True) + a_d
    m = jnp.maximum(mb, 0.2 * mb)
    adm_ref[:, :H] = a_d
    adm_ref[:, H:] = m
    e = a_s + a_d
    e = jnp.maximum(e, 0.2 * e)
    exself_ref[...] = jnp.exp(e - m)


def _prep(a_s, a_d):
    return pl.pallas_call(
        _prep_body,
        out_shape=[
            jax.ShapeDtypeStruct((N, 2 * H), jnp.float32),
            jax.ShapeDtypeStruct((N, H), jnp.float32),
        ],
    )(a_s, a_d)


def _tr_body(x_ref, o_ref):
    o_ref[...] = x_ref[...].T


def _tr(ex):
    be = 6400
    return pl.pallas_call(
        _tr_body,
        grid=(E // be,),
        in_specs=[pl.BlockSpec((be, H), lambda i: (i, 0))],
        out_specs=pl.BlockSpec((H, be), lambda i: (0, i)),
        out_shape=jax.ShapeDtypeStruct((H, E), jnp.float32),
    )(ex)


def _combine_body(acc_ref, ht_ref, exself_ref, den_ref, b_ref,
                  nm_ref, hm_ref):
    exs = exself_ref[...]
    den = den_ref[0] + den_ref[1] + exs + 1e-16
    for k in range(H):
        ek = exs[:, k:k + 1]
        r = jax.nn.relu(
            (acc_ref[k] + ek * ht_ref[k]) / den[:, k:k + 1] + b_ref[k])
        nm_ref[:, k, :] = r
        hm_ref[k] = r


def _combine(acc, ht, exself, den, b):
    return pl.pallas_call(
        _combine_body,
        grid=(NB,),
        in_specs=[
            pl.BlockSpec((H, BN, CP), lambda i: (0, i, 0)),
            pl.BlockSpec((H, BN, CP), lambda i: (0, i, 0)),
            pl.BlockSpec((BN, H), lambda i: (i, 0)),
            pl.BlockSpec((2, BN, H), lambda i: (0, i, 0)),
            pl.BlockSpec((H, 1, CP), lambda i: (0, 0, 0)),
        ],
        out_specs=[
            pl.BlockSpec((BN, H, CP), lambda i: (i, 0, 0)),
            pl.BlockSpec((H, BN, CP), lambda i: (0, i, 0)),
        ],
        out_shape=[
            jax.ShapeDtypeStruct((N, H, CP), jnp.float32),
            jax.ShapeDtypeStruct((H, N, CP), jnp.float32),
        ],
    )(acc, ht, exself, den, b)


def _graphconv_body(h2_ref, nbr_ref, wgr_ref, wgn_ref, bg_ref, o_ref):
    acc = jnp.dot(h2_ref[...], wgr_ref[...],
                  preferred_element_type=jnp.float32) + bg_ref[...]
    for k in range(H):
        acc += jnp.dot(nbr_ref[k], wgn_ref[k],
                       preferred_element_type=jnp.float32)
    o_ref[...] = jax.nn.relu(acc)


def _graphconv(h2n, nbrg, wgr, wgn, bg):
    return pl.pallas_call(
        _graphconv_body,
        grid=(NB,),
        in_specs=[
            pl.BlockSpec((BN, H * CP), lambda i: (i, 0)),
            pl.BlockSpec((H, BN, CP), lambda i: (0, i, 0)),
            pl.BlockSpec((H * CP, 48), lambda i: (0, 0)),
            pl.BlockSpec((H, CP, 48), lambda i: (0, 0, 0)),
            pl.BlockSpec((1, 48), lambda i: (0, 0)),
        ],
        out_specs=pl.BlockSpec((BN, 48), lambda i: (i, 0)),
        out_shape=jax.ShapeDtypeStruct((N, 48), jnp.float32),
    )(h2n, nbrg, wgr, wgn, bg)


def _head_body(h3_ref, nbr_ref, deg_ref, wsn_ref, wsr_ref, bs_ref,
               batch_ref, fc1w_ref, fc1b_ref, fc3w_ref, fc3b_ref,
               out_ref, sums_ref, cnt_ref):
    i = pl.program_id(0)

    @pl.when(i == 0)
    def _():
        sums_ref[...] = jnp.zeros_like(sums_ref)
        cnt_ref[...] = jnp.zeros_like(cnt_ref)

    nm = nbr_ref[...] / jnp.maximum(deg_ref[...], 1.0)
    h4 = jax.nn.relu(
        jnp.dot(nm, wsn_ref[...], preferred_element_type=jnp.float32)
        + jnp.dot(h3_ref[...], wsr_ref[...],
                  preferred_element_type=jnp.float32)
        + bs_ref[...])
    onehot = (batch_ref[...] == lax.broadcasted_iota(jnp.int32, (1, G), 1)
              ).astype(jnp.float32)
    sums_ref[...] += lax.dot_general(onehot, h4, (((0,), (0,)), ((), ())),
                                     preferred_element_type=jnp.float32)
    cnt_ref[...] += jnp.sum(onehot, axis=0, keepdims=True)

    @pl.when(i == NB - 1)
    def _():
        g = sums_ref[...] / jnp.maximum(cnt_ref[...], 1.0).T
        g = jax.nn.relu(g @ fc1w_ref[...] + fc1b_ref[...])
        out_ref[...] = g @ fc3w_ref[...] + fc3b_ref[...]


def _head(h3, nbr3, deg, wsn, wsr, bs, batch, fc1w, fc1b, fc3w, fc3b):
    return pl.pallas_call(
        _head_body,
        grid=(NB,),
        in_specs=[
            pl.BlockSpec((BN, 48), lambda i: (i, 0)),
            pl.BlockSpec((BN, 48), lambda i: (i, 0)),
            pl.BlockSpec((BN, 1), lambda i: (i, 0)),
            pl.BlockSpec((48, 48), lambda i: (0, 0)),
            pl.BlockSpec((48, 48), lambda i: (0, 0)),
            pl.BlockSpec((1, 48), lambda i: (0, 0)),
            pl.BlockSpec((BN, 1), lambda i: (i, 0)),
            pl.BlockSpec((48, 32), lambda i: (0, 0)),
            pl.BlockSpec((1, 32), lambda i: (0, 0)),
            pl.BlockSpec((32, 10), lambda i: (0, 0)),
            pl.BlockSpec((1, 10), lambda i: (0, 0)),
        ],
        out_specs=pl.BlockSpec((G, 10), lambda i: (0, 0)),
        out_shape=jax.ShapeDtypeStruct((G, 10), jnp.float32),
        scratch_shapes=[
            pltpu.VMEM((G, 48), jnp.float32),
            pltpu.VMEM((1, G), jnp.float32),
        ],
    )(h3, nbr3, deg, wsn, wsr, bs, batch, fc1w, fc1b, fc3w, fc3b)


def _sc_softmax_body(src_hbm, dst_hbm, as_hbm, adm_hbm, z_hbm,
                     ex_hbm, den_hbm, srcb, dstb, asb, admb, exb, den_sh):
    c = lax.axis_index("c")
    s = lax.axis_index("s")
    rpt = NP // NTILE
    r0 = s * rpt
    pltpu.sync_copy(z_hbm.at[pl.ds(r0, rpt)], den_sh.at[pl.ds(r0, rpt)])
    plsc.subcore_barrier()
    epc = E // 2
    ept = epc // NTILE

    @pl.loop(0, ept // CH)
    def _(ci):
        base = c * epc + s * ept + ci * CH
        pltpu.sync_copy(src_hbm.at[pl.ds(base, CH)], srcb)
        pltpu.sync_copy(dst_hbm.at[pl.ds(base, CH)], dstb)
        pltpu.sync_copy(as_hbm.at[srcb], asb)
        pltpu.sync_copy(adm_hbm.at[dstb], admb)

        @pl.loop(0, CH)
        def _(i):
            e = asb[i, :] + admb[i, pl.ds(0, H)]
            e = jnp.maximum(e, 0.2 * e)
            exb[i, :] = jnp.exp(e - admb[i, pl.ds(H, H)])

        pltpu.sync_copy(exb, ex_hbm.at[pl.ds(base, CH)])
        pltpu.sync_copy(exb, den_sh.at[dstb], add=True)

    plsc.subcore_barrier()

    @pl.when(c == 0)
    def _():
        pltpu.sync_copy(den_sh.at[pl.ds(r0, rpt)],
                        den_hbm.at[0, pl.ds(r0, rpt)])

    @pl.when(c == 1)
    def _():
        pltpu.sync_copy(den_sh.at[pl.ds(r0, rpt)],
                        den_hbm.at[1, pl.ds(r0, rpt)])


def _sc_softmax(src, dst, a_s, adm, z16):
    k = functools.partial(
        pl.kernel, mesh=_mesh, compiler_params=_sc_params,
        out_type=[jax.ShapeDtypeStruct((E, H), jnp.float32),
                  jax.ShapeDtypeStruct((2, NP, H), jnp.float32)],
        scratch_types=[
            pltpu.VMEM((CH,), jnp.int32),
            pltpu.VMEM((CH,), jnp.int32),
            pltpu.VMEM((CH, H), jnp.float32),
            pltpu.VMEM((CH, 2 * H), jnp.float32),
            pltpu.VMEM((CH, H), jnp.float32),
            pltpu.VMEM_SHARED((NP, H), jnp.float32),
        ])(_sc_softmax_body)
    return k(src, dst, a_s, adm, z16)


def _sc_gat_agg_body(ht_hbm, w_hbm, src_hbm, dst_hbm, z_hbm, out_hbm,
                     srcb, dstb, wb, rows, acc_sh):
    c = lax.axis_index("c")
    s = lax.axis_index("s")
    rpt = NP // NTILE
    r0 = s * rpt
    ept = E // NTILE
    for k in range(H):
        @pl.when(c == (k % 2))
        def _():
            pltpu.sync_copy(z_hbm.at[pl.ds(r0, rpt)],
                            acc_sh.at[pl.ds(r0, rpt)])
            plsc.subcore_barrier()

            @pl.loop(0, ept // CH)
            def _(ci):
                b = s * ept + ci * CH
                pltpu.sync_copy(src_hbm.at[pl.ds(b, CH)], srcb)
                pltpu.sync_copy(dst_hbm.at[pl.ds(b, CH)], dstb)
                pltpu.sync_copy(w_hbm.at[k, pl.ds(b, CH)], wb)
                pltpu.sync_copy(ht_hbm.at[k].at[srcb], rows)

                @pl.loop(0, CH)
                def _(i):
                    wv = plsc.load_gather(
                        wb, [jnp.full((16,), i, jnp.int32)])
                    for j in range(CP // 16):
                        rows[i, pl.ds(j * 16, 16)] = (
                            rows[i, pl.ds(j * 16, 16)] * wv)

                pltpu.sync_copy(rows, acc_sh.at[dstb], add=True)

            plsc.subcore_barrier()
            pltpu.sync_copy(acc_sh.at[pl.ds(r0, rpt)],
                            out_hbm.at[k, pl.ds(r0, rpt)])


def _sc_gat_agg(ht, w_t, src, dst, z64):
    k = functools.partial(
        pl.kernel, mesh=_mesh, compiler_params=_sc_params,
        out_type=jax.ShapeDtypeStruct((H, NP, CP), jnp.float32),
        scratch_types=[
            pltpu.VMEM((CH,), jnp.int32),
            pltpu.VMEM((CH,), jnp.int32),
            pltpu.VMEM((CH,), jnp.float32),
            pltpu.VMEM((CH, CP), jnp.float32),
            pltpu.VMEM_SHARED((NP, CP), jnp.float32),
        ])(_sc_gat_agg_body)
    return k(ht, w_t, src, dst, z64)


def _sc_sum_agg_body(ht_hbm, src_hbm, dst_hbm, z_hbm, out_hbm,
                     srcb, dstb, rows, acc_sh):
    c = lax.axis_index("c")
    s = lax.axis_index("s")
    rpt = NP // NTILE
    r0 = s * rpt
    ept = E // NTILE
    for k in range(H):
        @pl.when(c == (k % 2))
        def _():
            pltpu.sync_copy(z_hbm.at[pl.ds(r0, rpt)],
                            acc_sh.at[pl.ds(r0, rpt)])
            plsc.subcore_barrier()

            @pl.loop(0, ept // CH)
            def _(ci):
                b = s * ept + ci * CH
                pltpu.sync_copy(src_hbm.at[pl.ds(b, CH)], srcb)
                pltpu.sync_copy(dst_hbm.at[pl.ds(b, CH)], dstb)
                pltpu.sync_copy(ht_hbm.at[k].at[srcb], rows)
                pltpu.sync_copy(rows, acc_sh.at[dstb], add=True)

            plsc.subcore_barrier()
            pltpu.sync_copy(acc_sh.at[pl.ds(r0, rpt)],
                            out_hbm.at[k, pl.ds(r0, rpt)])


def _sc_sum_agg(ht, src, dst, z64):
    k = functools.partial(
        pl.kernel, mesh=_mesh, compiler_params=_sc_params,
        out_type=jax.ShapeDtypeStruct((H, NP, CP), jnp.float32),
        scratch_types=[
            pltpu.VMEM((CH,), jnp.int32),
            pltpu.VMEM((CH,), jnp.int32),
            pltpu.VMEM((CH, CP), jnp.float32),
            pltpu.VMEM_SHARED((NP, CP), jnp.float32),
        ])(_sc_sum_agg_body)
    return k(ht, src, dst, z64)


def _sc_sage_body(h3_hbm, src_hbm, dst_hbm, ones_hbm, z48_hbm, z16_hbm,
                  nbr_hbm, deg_hbm, srcb, dstb, rows, onesb,
                  nbr_sh, deg_sh):
    c = lax.axis_index("c")
    s = lax.axis_index("s")
    rpt = NP // NTILE
    r0 = s * rpt
    ept = E // NTILE

    @pl.when(c == 0)
    def _():
        pltpu.sync_copy(z48_hbm.at[pl.ds(r0, rpt)],
                        nbr_sh.at[pl.ds(r0, rpt)])
        plsc.subcore_barrier()

        @pl.loop(0, ept // CH)
        def _(ci):
            b = s * ept + ci * CH
            pltpu.sync_copy(src_hbm.at[pl.ds(b, CH)], srcb)
            pltpu.sync_copy(dst_hbm.at[pl.ds(b, CH)], dstb)
            pltpu.sync_copy(h3_hbm.at[srcb], rows)
            pltpu.sync_copy(rows, nbr_sh.at[dstb], add=True)

        plsc.subcore_barrier()
        pltpu.sync_copy(nbr_sh.at[pl.ds(r0, rpt)],
                        nbr_hbm.at[pl.ds(r0, rpt)])

    @pl.when(c == 1)
    def _():
        pltpu.sync_copy(z16_hbm.at[pl.ds(r0, rpt)],
                        deg_sh.at[pl.ds(r0, rpt)])
        pltpu.sync_copy(ones_hbm, onesb)
        plsc.subcore_barrier()

        @pl.loop(0, ept // CH)
        def _(ci):
            b = s * ept + ci * CH
            pltpu.sync_copy(dst_hbm.at[pl.ds(b, CH)], dstb)
            pltpu.sync_copy(onesb, deg_sh.at[dstb], add=True)

        plsc.subcore_barrier()
        pltpu.sync_copy(deg_sh.at[pl.ds(r0, rpt)],
                        deg_hbm.at[pl.ds(r0, rpt)])


def _sc_sage(h3, src, dst, ones, z48, z16):
    k = functools.partial(
        pl.kernel, mesh=_mesh, compiler_params=_sc_params,
        out_type=[jax.ShapeDtypeStruct((NP, 48), jnp.float32),
                  jax.ShapeDtypeStruct((NP, H), jnp.float32)],
        scratch_types=[
            pltpu.VMEM((CH,), jnp.int32),
            pltpu.VMEM((CH,), jnp.int32),
            pltpu.VMEM((CH, 48), jnp.float32),
            pltpu.VMEM((CH, H), jnp.float32),
            pltpu.VMEM_SHARED((N, 48), jnp.float32),
            pltpu.VMEM_SHARED((NP, H), jnp.float32),
        ])(_sc_sage_body)
    return k(h3, src, dst, ones, z48, z16)


def _gat_layer(x, src, dst, wt, asr, adr, bp, z16, z64):
    ht = _mm(x, wt)
    a_s, a_d = _asad(ht, asr, adr)
    adm, exself = _prep(a_s, a_d)
    ex, den = _sc_softmax(src, dst, a_s, adm, z16)
    ex_t = _tr(ex)
    acc = _sc_gat_agg(ht, ex_t, src, dst, z64)
    return _combine(acc, ht, exself, den, bp)


def kernel(x, edge_index, batch, W1, att_src1, att_dst1, b1, W2, att_src2,
           att_dst2, b2, Wg_root, Wg_nbr, bg, Ws_nbr, Ws_root, bs, fc1_w,
           fc1_b, fc3_w, fc3_b):
    f32 = jnp.float32
    src = edge_index[0]
    dst = edge_index[1]
    C1 = att_src1.shape[1]
    C2 = att_src2.shape[1]

    def padh(a, c):
        return jnp.pad(a.reshape(H, 1, c), ((0, 0), (0, 0), (0, CP - c)))

    w1t = jnp.pad(W1.reshape(-1, H, C1), ((0, 0), (0, 0), (0, CP - C1))
                  ).transpose(1, 0, 2)
    w2t = jnp.pad(W2.reshape(H, C1, H, C2),
                  ((0, 0), (0, CP - C1), (0, 0), (0, CP - C2))
                  ).reshape(H * CP, H, CP).transpose(1, 0, 2)
    wgr = jnp.pad(Wg_root.reshape(H, C2, 40),
                  ((0, 0), (0, CP - C2), (0, 8))).reshape(H * CP, 48)
    wgn = jnp.pad(Wg_nbr.reshape(H, C2, 40),
                  ((0, 0), (0, CP - C2), (0, 8)))
    wsn = jnp.pad(Ws_nbr, ((0, 8), (0, 8)))
    wsr = jnp.pad(Ws_root, ((0, 8), (0, 8)))
    fc1p = jnp.pad(fc1_w, ((0, 8), (0, 0)))

    z16 = jnp.zeros((NP, H), f32)
    z48 = jnp.zeros((NP, 48), f32)
    z64 = jnp.zeros((NP, CP), f32)
    ones = jnp.ones((CH, H), f32)

    h2nm, _ = _gat_layer(x, src, dst, w1t, padh(att_src1, C1),
                         padh(att_dst1, C1), padh(b1, C1), z16, z64)
    h2 = h2nm.reshape(N, H * CP)
    h3nm, h3hm = _gat_layer(h2, src, dst, w2t, padh(att_src2, C2),
                            padh(att_dst2, C2), padh(b2, C2), z16, z64)
    h3n = h3nm.reshape(N, H * CP)

    nbrg = _sc_sum_agg(h3hm, src, dst, z64)
    h4 = _graphconv(h3n, nbrg, wgr, wgn, jnp.pad(bg, (0, 8)).reshape(1, 48))

    nbr3, deg = _sc_sage(h4, src, dst, ones, z48, z16)
    out = _head(h4, nbr3, deg[:, :1], wsn, wsr,
                jnp.pad(bs, (0, 8)).reshape(1, 48), batch.reshape(N, 1),
                fc1p, fc1_b.reshape(1, 32), fc3_w, fc3_b.reshape(1, 10))
    return out


# speedup vs baseline: 13.3721x; 1.1480x over previous
import functools

import jax
import jax.numpy as jnp
from jax import lax
from jax.experimental import pallas as pl
from jax.experimental.pallas import tpu as pltpu
from jax.experimental.pallas import tpu_sc as plsc

N = 10000
E = 320000
G = 64
H = 16
CP = 64
BN = 400
NB = N // BN
NP = 10240
NTILE = 16
CH = 1000
CHB = 400
_mesh = plsc.VectorSubcoreMesh(core_axis_name="c", subcore_axis_name="s")
_sc_params = pltpu.CompilerParams(use_tc_tiling_on_sc=False, needs_layout_passes=False)


def _mm_body(x_ref, w_ref, ht_ref):
    ht_ref[0] = jnp.dot(x_ref[...], w_ref[0],
                        preferred_element_type=jnp.float32)


def _mm(x, wt):
    d = x.shape[1]
    return pl.pallas_call(
        _mm_body,
        grid=(NB, H),
        in_specs=[
            pl.BlockSpec((BN, d), lambda i, k: (i, 0)),
            pl.BlockSpec((1, d, CP), lambda i, k: (k, 0, 0)),
        ],
        out_specs=pl.BlockSpec((1, BN, CP), lambda i, k: (k, i, 0)),
        out_shape=jax.ShapeDtypeStruct((H, N, CP), jnp.float32),
    )(x, wt)


def _asad_body(ht_ref, asr_ref, adr_ref, as_ref, ad_ref):
    cols_s = []
    cols_d = []
    for k in range(H):
        h = ht_ref[k]
        cols_s.append(jnp.sum(h * asr_ref[k], axis=1, keepdims=True))
        cols_d.append(jnp.sum(h * adr_ref[k], axis=1, keepdims=True))
    as_ref[...] = jnp.concatenate(cols_s, axis=1)
    ad_ref[...] = jnp.concatenate(cols_d, axis=1)


def _asad(ht, asr, adr):
    return pl.pallas_call(
        _asad_body,
        grid=(NB,),
        in_specs=[
            pl.BlockSpec((H, BN, CP), lambda i: (0, i, 0)),
            pl.BlockSpec((H, 1, CP), lambda i: (0, 0, 0)),
            pl.BlockSpec((H, 1, CP), lambda i: (0, 0, 0)),
        ],
        out_specs=[
            pl.BlockSpec((BN, H), lambda i: (i, 0)),
            pl.BlockSpec((BN, H), lambda i: (i, 0)),
        ],
        out_shape=[
            jax.ShapeDtypeStruct((N, H), jnp.float32),
            jax.ShapeDtypeStruct((N, H), jnp.float32),
        ],
    )(ht, asr, adr)


def _prep_body(as_ref, ad_ref, adm_ref, exself_ref):
    a_s = as_ref[...]
    a_d = ad_ref[...]
    mb = jnp.max(a_s, axis=0, keepdims=True) + a_d
    m = jnp.maximum(mb, 0.2 * mb)
    adm_ref[:, :H] = a_d
    adm_ref[:, H:] = m
    e = a_s + a_d
    e = jnp.maximum(e, 0.2 * e)
    exself_ref[...] = jnp.exp(e - m)


def _prep(a_s, a_d):
    return pl.pallas_call(
        _prep_body,
        out_shape=[
            jax.ShapeDtypeStruct((N, 2 * H), jnp.float32),
            jax.ShapeDtypeStruct((N, H), jnp.float32),
        ],
    )(a_s, a_d)


def _tr_body(x_ref, o_ref):
    o_ref[...] = x_ref[...].T


def _tr(ex):
    be = 6400
    return pl.pallas_call(
        _tr_body,
        grid=(E // be,),
        in_specs=[pl.BlockSpec((be, H), lambda i: (i, 0))],
        out_specs=pl.BlockSpec((H, be), lambda i: (0, i)),
        out_shape=jax.ShapeDtypeStruct((H, E), ex.dtype),
    )(ex)


def _scale_body(ht_ref, as_ref, t2_ref):
    a_s = as_ref[...]
    for k in range(H):
        h = ht_ref[k]
        t2_ref[k, 0] = h * jnp.exp(a_s[:, k:k + 1])
        t2_ref[k, 1] = h * jnp.exp(0.2 * a_s[:, k:k + 1])


def _scale_tables(ht, a_s):
    return pl.pallas_call(
        _scale_body,
        grid=(NB,),
        in_specs=[
            pl.BlockSpec((H, BN, CP), lambda i: (0, i, 0)),
            pl.BlockSpec((BN, H), lambda i: (i, 0)),
        ],
        out_specs=pl.BlockSpec((H, 2, BN, CP), lambda i: (0, 0, i, 0)),
        out_shape=jax.ShapeDtypeStruct((H, 2, NP, CP), jnp.float32),
    )(ht, a_s)


def _combine_body(accp_ref, accn_ref, ht_ref, exself_ref, den_ref,
                  adm_ref, b_ref, nm_ref, hm_ref):
    exs = exself_ref[...]
    adm = adm_ref[...]
    den = den_ref[0] + den_ref[1] + exs + 1e-16
    for k in range(H):
        ad_k = adm[:, k:k + 1]
        m_k = adm[:, H + k:H + k + 1]
        num = (jnp.exp(ad_k - m_k) * accp_ref[k, 0]
               + jnp.exp(0.2 * ad_k - m_k) * accn_ref[k, 0]
               + exs[:, k:k + 1] * ht_ref[k])
        r = jax.nn.relu(num / den[:, k:k + 1] + b_ref[k])
        nm_ref[:, k, :] = r
        hm_ref[k] = r


def _combine(acc, ht, exself, den, adm, b):
    return pl.pallas_call(
        _combine_body,
        grid=(NB,),
        in_specs=[
            pl.BlockSpec((H, 1, BN, CP), lambda i: (0, 0, i, 0)),
            pl.BlockSpec((H, 1, BN, CP), lambda i: (0, 1, i, 0)),
            pl.BlockSpec((H, BN, CP), lambda i: (0, i, 0)),
            pl.BlockSpec((BN, H), lambda i: (i, 0)),
            pl.BlockSpec((2, BN, H), lambda i: (0, i, 0)),
            pl.BlockSpec((BN, 2 * H), lambda i: (i, 0)),
            pl.BlockSpec((H, 1, CP), lambda i: (0, 0, 0)),
        ],
        out_specs=[
            pl.BlockSpec((BN, H, CP), lambda i: (i, 0, 0)),
            pl.BlockSpec((H, BN, CP), lambda i: (0, i, 0)),
        ],
        out_shape=[
            jax.ShapeDtypeStruct((N, H, CP), jnp.float32),
            jax.ShapeDtypeStruct((H, N, CP), jnp.float32),
        ],
    )(acc, acc, ht, exself, den, adm, b)


def _graphconv_body(h2_ref, nbr_ref, wgr_ref, wgn_ref, bg_ref, o_ref):
    acc = jnp.dot(h2_ref[...], wgr_ref[...],
                  preferred_element_type=jnp.float32) + bg_ref[...]
    for k in range(H):
        acc += jnp.dot(nbr_ref[k], wgn_ref[k],
                       preferred_element_type=jnp.float32)
    o_ref[...] = jax.nn.relu(acc)


def _graphconv(h2n, nbrg, wgr, wgn, bg):
    return pl.pallas_call(
        _graphconv_body,
        grid=(NB,),
        in_specs=[
            pl.BlockSpec((BN, H * CP), lambda i: (i, 0)),
            pl.BlockSpec((H, BN, CP), lambda i: (0, i, 0)),
            pl.BlockSpec((H * CP, 48), lambda i: (0, 0)),
            pl.BlockSpec((H, CP, 48), lambda i: (0, 0, 0)),
            pl.BlockSpec((1, 48), lambda i: (0, 0)),
        ],
        out_specs=pl.BlockSpec((BN, 48), lambda i: (i, 0)),
        out_shape=jax.ShapeDtypeStruct((N, 48), jnp.float32),
    )(h2n, nbrg, wgr, wgn, bg)


def _head_body(h3_ref, nbr_ref, deg_ref, wsn_ref, wsr_ref, bs_ref,
               batch_ref, fc1w_ref, fc1b_ref, fc3w_ref, fc3b_ref,
               out_ref, sums_ref, cnt_ref):
    i = pl.program_id(0)

    @pl.when(i == 0)
    def _():
        sums_ref[...] = jnp.zeros_like(sums_ref)
        cnt_ref[...] = jnp.zeros_like(cnt_ref)

    nm = nbr_ref[...] / jnp.maximum(deg_ref[...], 1.0)
    h4 = jax.nn.relu(
        jnp.dot(nm, wsn_ref[...], preferred_element_type=jnp.float32)
        + jnp.dot(h3_ref[...], wsr_ref[...],
                  preferred_element_type=jnp.float32)
        + bs_ref[...])
    onehot = (batch_ref[...] == lax.broadcasted_iota(jnp.int32, (1, G), 1)
              ).astype(jnp.float32)
    sums_ref[...] += lax.dot_general(onehot, h4, (((0,), (0,)), ((), ())),
                                     preferred_element_type=jnp.float32)
    cnt_ref[...] += jnp.sum(onehot, axis=0, keepdims=True)

    @pl.when(i == NB - 1)
    def _():
        g = sums_ref[...] / jnp.maximum(cnt_ref[...], 1.0).T
        g = jax.nn.relu(g @ fc1w_ref[...] + fc1b_ref[...])
        out_ref[...] = g @ fc3w_ref[...] + fc3b_ref[...]


def _head(h3, nbr3, deg, wsn, wsr, bs, batch, fc1w, fc1b, fc3w, fc3b):
    return pl.pallas_call(
        _head_body,
        grid=(NB,),
        in_specs=[
            pl.BlockSpec((BN, 48), lambda i: (i, 0)),
            pl.BlockSpec((BN, 48), lambda i: (i, 0)),
            pl.BlockSpec((BN, 1), lambda i: (i, 0)),
            pl.BlockSpec((48, 48), lambda i: (0, 0)),
            pl.BlockSpec((48, 48), lambda i: (0, 0)),
            pl.BlockSpec((1, 48), lambda i: (0, 0)),
            pl.BlockSpec((BN, 1), lambda i: (i, 0)),
            pl.BlockSpec((48, 32), lambda i: (0, 0)),
            pl.BlockSpec((1, 32), lambda i: (0, 0)),
            pl.BlockSpec((32, 10), lambda i: (0, 0)),
            pl.BlockSpec((1, 10), lambda i: (0, 0)),
        ],
        out_specs=pl.BlockSpec((G, 10), lambda i: (0, 0)),
        out_shape=jax.ShapeDtypeStruct((G, 10), jnp.float32),
        scratch_shapes=[
            pltpu.VMEM((G, 48), jnp.float32),
            pltpu.VMEM((1, G), jnp.float32),
        ],
    )(h3, nbr3, deg, wsn, wsr, bs, batch, fc1w, fc1b, fc3w, fc3b)


def _sc_softmax_body(src_hbm, dst_hbm, as_hbm, adm_hbm, z_hbm,
                     den_hbm, gidx_hbm, sidx_hbm,
                     srcb, dstb, asb, admb, exb, gib, sib, den_sh):
    c = lax.axis_index("c")
    s = lax.axis_index("s")
    rpt = NP // NTILE
    r0 = s * rpt
    pltpu.sync_copy(z_hbm.at[pl.ds(r0, rpt)], den_sh.at[pl.ds(r0, rpt)])
    plsc.subcore_barrier()
    epc = E // 2
    ept = epc // NTILE

    @pl.loop(0, ept // CH)
    def _(ci):
        base = c * epc + s * ept + ci * CH
        pltpu.sync_copy(src_hbm.at[pl.ds(base, CH)], srcb)
        pltpu.sync_copy(dst_hbm.at[pl.ds(base, CH)], dstb)
        pltpu.sync_copy(as_hbm.at[srcb], asb)
        pltpu.sync_copy(adm_hbm.at[dstb], admb)

        @pl.loop(0, CH)
        def _(i):
            ei = asb[i, :] + admb[i, pl.ds(0, H)]
            e = jnp.maximum(ei, 0.2 * ei)
            exb[i, :] = jnp.exp(e - admb[i, pl.ds(H, H)])
            off = jnp.where(ei < 0.0, NP, 0).astype(jnp.int32)
            srcv = plsc.load_gather(srcb, [jnp.full((16,), i, jnp.int32)])
            dstv = plsc.load_gather(dstb, [jnp.full((16,), i, jnp.int32)])
            gib[i, :] = srcv + off
            sib[i, :] = dstv + off

        pltpu.sync_copy(exb, den_sh.at[dstb], add=True)
        pltpu.sync_copy(gib, gidx_hbm.at[pl.ds(base, CH)])
        pltpu.sync_copy(sib, sidx_hbm.at[pl.ds(base, CH)])

    plsc.subcore_barrier()

    @pl.when(c == 0)
    def _():
        pltpu.sync_copy(den_sh.at[pl.ds(r0, rpt)],
                        den_hbm.at[0, pl.ds(r0, rpt)])

    @pl.when(c == 1)
    def _():
        pltpu.sync_copy(den_sh.at[pl.ds(r0, rpt)],
                        den_hbm.at[1, pl.ds(r0, rpt)])


def _sc_softmax(src, dst, a_s, adm, z16):
    k = functools.partial(
        pl.kernel, mesh=_mesh, compiler_params=_sc_params,
        out_type=[jax.ShapeDtypeStruct((2, NP, H), jnp.float32),
                  jax.ShapeDtypeStruct((E, H), jnp.int32),
                  jax.ShapeDtypeStruct((E, H), jnp.int32)],
        scratch_types=[
            pltpu.VMEM((CH,), jnp.int32),
            pltpu.VMEM((CH,), jnp.int32),
            pltpu.VMEM((CH, H), jnp.float32),
            pltpu.VMEM((CH, 2 * H), jnp.float32),
            pltpu.VMEM((CH, H), jnp.float32),
            pltpu.VMEM((CH, H), jnp.int32),
            pltpu.VMEM((CH, H), jnp.int32),
            pltpu.VMEM_SHARED((NP, H), jnp.float32),
        ])(_sc_softmax_body)
    return k(src, dst, a_s, adm, z16)


def _sc_gat_agg_body(t2_hbm, gidx_hbm, sidx_hbm, z_hbm, out_hbm,
                     gib, sib, rows, acc_sh):
    c = lax.axis_index("c")
    s = lax.axis_index("s")
    rpt = (2 * NP) // NTILE
    r0 = s * rpt
    ept = E // NTILE
    for k in range(H):
        @pl.when(c == (k % 2))
        def _():
            pltpu.sync_copy(z_hbm.at[pl.ds(r0, rpt)],
                            acc_sh.at[pl.ds(r0, rpt)])
            plsc.subcore_barrier()

            @pl.loop(0, ept // CHB)
            def _(ci):
                b = s * ept + ci * CHB
                pltpu.sync_copy(gidx_hbm.at[k, pl.ds(b, CHB)], gib)
                pltpu.sync_copy(sidx_hbm.at[k, pl.ds(b, CHB)], sib)
                pltpu.sync_copy(t2_hbm.at[k].at[gib], rows)
                pltpu.sync_copy(rows, acc_sh.at[sib], add=True)

            plsc.subcore_barrier()
            pltpu.sync_copy(acc_sh.at[pl.ds(r0, rpt)],
                            out_hbm.at[k, pl.ds(r0, rpt)])


def _sc_gat_agg(t2, gidx_t, sidx_t, z2):
    k = functools.partial(
        pl.kernel, mesh=_mesh, compiler_params=_sc_params,
        out_type=jax.ShapeDtypeStruct((H, 2 * NP, CP), jnp.float32),
        scratch_types=[
            pltpu.VMEM((CHB,), jnp.int32),
            pltpu.VMEM((CHB,), jnp.int32),
            pltpu.VMEM((CHB, CP), jnp.float32),
            pltpu.VMEM_SHARED((2 * NP, CP), jnp.float32),
        ])(_sc_gat_agg_body)
    return k(t2, gidx_t, sidx_t, z2)


def _sc_sum_agg_body(ht_hbm, src_hbm, dst_hbm, z_hbm, out_hbm,
                     srcb, dstb, rows, acc_sh):
    c = lax.axis_index("c")
    s = lax.axis_index("s")
    rpt = NP // NTILE
    r0 = s * rpt
    ept = E // NTILE
    for k in range(H):
        @pl.when(c == (k % 2))
        def _():
            pltpu.sync_copy(z_hbm.at[pl.ds(r0, rpt)],
                            acc_sh.at[pl.ds(r0, rpt)])
            plsc.subcore_barrier()

            @pl.loop(0, ept // CH)
            def _(ci):
                b = s * ept + ci * CH
                pltpu.sync_copy(src_hbm.at[pl.ds(b, CH)], srcb)
                pltpu.sync_copy(dst_hbm.at[pl.ds(b, CH)], dstb)
                pltpu.sync_copy(ht_hbm.at[k].at[srcb], rows)
                pltpu.sync_copy(rows, acc_sh.at[dstb], add=True)

            plsc.subcore_barrier()
            pltpu.sync_copy(acc_sh.at[pl.ds(r0, rpt)],
                            out_hbm.at[k, pl.ds(r0, rpt)])


def _sc_sum_agg(ht, src, dst, z64):
    k = functools.partial(
        pl.kernel, mesh=_mesh, compiler_params=_sc_params,
        out_type=jax.ShapeDtypeStruct((H, NP, CP), jnp.float32),
        scratch_types=[
            pltpu.VMEM((CH,), jnp.int32),
            pltpu.VMEM((CH,), jnp.int32),
            pltpu.VMEM((CH, CP), jnp.float32),
            pltpu.VMEM_SHARED((NP, CP), jnp.float32),
        ])(_sc_sum_agg_body)
    return k(ht, src, dst, z64)


def _sc_sage_body(h3_hbm, src_hbm, dst_hbm, ones_hbm, z48_hbm, z16_hbm,
                  nbr_hbm, deg_hbm, srcb, dstb, rows, onesb,
                  nbr_sh, deg_sh):
    c = lax.axis_index("c")
    s = lax.axis_index("s")
    rpt = NP // NTILE
    r0 = s * rpt
    ept = E // NTILE

    @pl.when(c == 0)
    def _():
        pltpu.sync_copy(z48_hbm.at[pl.ds(r0, rpt)],
                        nbr_sh.at[pl.ds(r0, rpt)])
        plsc.subcore_barrier()

        @pl.loop(0, ept // CH)
        def _(ci):
            b = s * ept + ci * CH
            pltpu.sync_copy(src_hbm.at[pl.ds(b, CH)], srcb)
            pltpu.sync_copy(dst_hbm.at[pl.ds(b, CH)], dstb)
            pltpu.sync_copy(h3_hbm.at[srcb], rows)
            pltpu.sync_copy(rows, nbr_sh.at[dstb], add=True)

        plsc.subcore_barrier()
        pltpu.sync_copy(nbr_sh.at[pl.ds(r0, rpt)],
                        nbr_hbm.at[pl.ds(r0, rpt)])

    @pl.when(c == 1)
    def _():
        pltpu.sync_copy(z16_hbm.at[pl.ds(r0, rpt)],
                        deg_sh.at[pl.ds(r0, rpt)])
        pltpu.sync_copy(ones_hbm, onesb)
        plsc.subcore_barrier()

        @pl.loop(0, ept // CH)
        def _(ci):
            b = s * ept + ci * CH
            pltpu.sync_copy(dst_hbm.at[pl.ds(b, CH)], dstb)
            pltpu.sync_copy(onesb, deg_sh.at[dstb], add=True)

        plsc.subcore_barrier()
        pltpu.sync_copy(deg_sh.at[pl.ds(r0, rpt)],
                        deg_hbm.at[pl.ds(r0, rpt)])


def _sc_sage(h3, src, dst, ones, z48, z16):
    k = functools.partial(
        pl.kernel, mesh=_mesh, compiler_params=_sc_params,
        out_type=[jax.ShapeDtypeStruct((NP, 48), jnp.float32),
                  jax.ShapeDtypeStruct((NP, H), jnp.float32)],
        scratch_types=[
            pltpu.VMEM((CH,), jnp.int32),
            pltpu.VMEM((CH,), jnp.int32),
            pltpu.VMEM((CH, 48), jnp.float32),
            pltpu.VMEM((CH, H), jnp.float32),
            pltpu.VMEM_SHARED((N, 48), jnp.float32),
            pltpu.VMEM_SHARED((NP, H), jnp.float32),
        ])(_sc_sage_body)
    return k(h3, src, dst, ones, z48, z16)


def _gat_layer(x, src, dst, wt, asr, adr, bp, z16, z2):
    ht = _mm(x, wt)
    a_s, a_d = _asad(ht, asr, adr)
    adm, exself = _prep(a_s, a_d)
    den, gidx, sidx = _sc_softmax(src, dst, a_s, adm, z16)
    gidx_t = _tr(gidx)
    sidx_t = _tr(sidx)
    t2 = _scale_tables(ht, a_s).reshape(H, 2 * NP, CP)
    acc = _sc_gat_agg(t2, gidx_t, sidx_t, z2).reshape(H, 2, NP, CP)
    return _combine(acc, ht, exself, den, adm, bp)


def kernel(x, edge_index, batch, W1, att_src1, att_dst1, b1, W2, att_src2,
           att_dst2, b2, Wg_root, Wg_nbr, bg, Ws_nbr, Ws_root, bs, fc1_w,
           fc1_b, fc3_w, fc3_b):
    f32 = jnp.float32
    src = edge_index[0]
    dst = edge_index[1]
    C1 = att_src1.shape[1]
    C2 = att_src2.shape[1]

    def padh(a, c):
        return jnp.pad(a.reshape(H, 1, c), ((0, 0), (0, 0), (0, CP - c)))

    w1t = jnp.pad(W1.reshape(-1, H, C1), ((0, 0), (0, 0), (0, CP - C1))
                  ).transpose(1, 0, 2)
    w2t = jnp.pad(W2.reshape(H, C1, H, C2),
                  ((0, 0), (0, CP - C1), (0, 0), (0, CP - C2))
                  ).reshape(H * CP, H, CP).transpose(1, 0, 2)
    wgr = jnp.pad(Wg_root.reshape(H, C2, 40),
                  ((0, 0), (0, CP - C2), (0, 8))).reshape(H * CP, 48)
    wgn = jnp.pad(Wg_nbr.reshape(H, C2, 40),
                  ((0, 0), (0, CP - C2), (0, 8)))
    wsn = jnp.pad(Ws_nbr, ((0, 8), (0, 8)))
    wsr = jnp.pad(Ws_root, ((0, 8), (0, 8)))
    fc1p = jnp.pad(fc1_w, ((0, 8), (0, 0)))

    z16 = jnp.zeros((NP, H), f32)
    z2 = jnp.zeros((2 * NP, CP), f32)
    z48 = jnp.zeros((NP, 48), f32)
    z64 = jnp.zeros((NP, CP), f32)
    ones = jnp.ones((CH, H), f32)

    h2nm, _ = _gat_layer(x, src, dst, w1t, padh(att_src1, C1),
                         padh(att_dst1, C1), padh(b1, C1), z16, z2)
    h2 = h2nm.reshape(N, H * CP)
    h3nm, h3hm = _gat_layer(h2, src, dst, w2t, padh(att_src2, C2),
                            padh(att_dst2, C2), padh(b2, C2), z16, z2)
    h3n = h3nm.reshape(N, H * CP)

    nbrg = _sc_sum_agg(h3hm, src, dst, z64)
    h4 = _graphconv(h3n, nbrg, wgr, wgn, jnp.pad(bg, (0, 8)).reshape(1, 48))

    nbr3, deg = _sc_sage(h4, src, dst, ones, z48, z16)
    out = _head(h4, nbr3, deg[:, :1], wsn, wsr,
                jnp.pad(bs, (0, 8)).reshape(1, 48), batch.reshape(N, 1),
                fc1p, fc1_b.reshape(1, 32), fc3_w, fc3_b.reshape(1, 10))
    return out


# speedup vs baseline: 17.5131x; 1.3097x over previous
import functools

import jax
import jax.numpy as jnp
from jax import lax
from jax.experimental import pallas as pl
from jax.experimental.pallas import tpu as pltpu
from jax.experimental.pallas import tpu_sc as plsc

N = 10000
E = 320000
G = 64
H = 16
CP = 64
BN = 400
NB = N // BN
NP = 10240
NTILE = 16
CH = 1000
CHB = 400
_mesh = plsc.VectorSubcoreMesh(core_axis_name="c", subcore_axis_name="s")
_sc_params = pltpu.CompilerParams(use_tc_tiling_on_sc=False, needs_layout_passes=False)


def _mm_body(x_ref, w_ref, ht_ref):
    ht_ref[0] = jnp.dot(x_ref[...], w_ref[0],
                        preferred_element_type=jnp.float32)


def _mm(x, wt):
    d = x.shape[1]
    return pl.pallas_call(
        _mm_body,
        grid=(NB, H),
        in_specs=[
            pl.BlockSpec((BN, d), lambda i, k: (i, 0)),
            pl.BlockSpec((1, d, CP), lambda i, k: (k, 0, 0)),
        ],
        out_specs=pl.BlockSpec((1, BN, CP), lambda i, k: (k, i, 0)),
        out_shape=jax.ShapeDtypeStruct((H, N, CP), jnp.float32),
    )(x, wt)


def _asad_body(ht_ref, asr_ref, adr_ref, as_ref, ad_ref):
    cols_s = []
    cols_d = []
    for k in range(H):
        h = ht_ref[k]
        cols_s.append(jnp.sum(h * asr_ref[k], axis=1, keepdims=True))
        cols_d.append(jnp.sum(h * adr_ref[k], axis=1, keepdims=True))
    as_ref[...] = jnp.concatenate(cols_s, axis=1)
    ad_ref[...] = jnp.concatenate(cols_d, axis=1)


def _asad(ht, asr, adr):
    return pl.pallas_call(
        _asad_body,
        grid=(NB,),
        in_specs=[
            pl.BlockSpec((H, BN, CP), lambda i: (0, i, 0)),
            pl.BlockSpec((H, 1, CP), lambda i: (0, 0, 0)),
            pl.BlockSpec((H, 1, CP), lambda i: (0, 0, 0)),
        ],
        out_specs=[
            pl.BlockSpec((BN, H), lambda i: (i, 0)),
            pl.BlockSpec((BN, H), lambda i: (i, 0)),
        ],
        out_shape=[
            jax.ShapeDtypeStruct((N, H), jnp.float32),
            jax.ShapeDtypeStruct((N, H), jnp.float32),
        ],
    )(ht, asr, adr)


def _prep_body(as_ref, ad_ref, adm_ref, exself_ref):
    a_s = as_ref[...]
    a_d = ad_ref[...]
    mb = jnp.max(a_s, axis=0, keepdims=True) + a_d
    m = jnp.maximum(mb, 0.2 * mb)
    adm_ref[:, :H] = a_d
    adm_ref[:, H:] = m
    e = a_s + a_d
    e = jnp.maximum(e, 0.2 * e)
    exself_ref[...] = jnp.exp(e - m)


def _prep(a_s, a_d):
    return pl.pallas_call(
        _prep_body,
        out_shape=[
            jax.ShapeDtypeStruct((N, 2 * H), jnp.float32),
            jax.ShapeDtypeStruct((N, H), jnp.float32),
        ],
    )(a_s, a_d)


def _tr_body(x_ref, o_ref):
    o_ref[...] = x_ref[...].T


def _tr(ex):
    be = 6400
    return pl.pallas_call(
        _tr_body,
        grid=(E // be,),
        in_specs=[pl.BlockSpec((be, H), lambda i: (i, 0))],
        out_specs=pl.BlockSpec((H, be), lambda i: (0, i)),
        out_shape=jax.ShapeDtypeStruct((H, E), ex.dtype),
    )(ex)


def _scale_body(ht_ref, as_ref, t2_ref):
    a_s = as_ref[...]
    for k in range(H):
        h = ht_ref[k]
        t2_ref[k, 0] = h * jnp.exp(a_s[:, k:k + 1])
        t2_ref[k, 1] = h * jnp.exp(0.2 * a_s[:, k:k + 1])


def _scale_tables(ht, a_s):
    return pl.pallas_call(
        _scale_body,
        grid=(NB,),
        in_specs=[
            pl.BlockSpec((H, BN, CP), lambda i: (0, i, 0)),
            pl.BlockSpec((BN, H), lambda i: (i, 0)),
        ],
        out_specs=pl.BlockSpec((H, 2, BN, CP), lambda i: (0, 0, i, 0)),
        out_shape=jax.ShapeDtypeStruct((H, 2, NP, CP), jnp.float32),
    )(ht, a_s)


def _combine_body(accp_ref, accn_ref, ht_ref, exself_ref, den_ref,
                  adm_ref, b_ref, nm_ref, hm_ref):
    exs = exself_ref[...]
    adm = adm_ref[...]
    den = den_ref[0] + den_ref[1] + exs + 1e-16
    for k in range(H):
        ad_k = adm[:, k:k + 1]
        m_k = adm[:, H + k:H + k + 1]
        num = (jnp.exp(ad_k - m_k) * accp_ref[k, 0]
               + jnp.exp(0.2 * ad_k - m_k) * accn_ref[k, 0]
               + exs[:, k:k + 1] * ht_ref[k])
        r = jax.nn.relu(num / den[:, k:k + 1] + b_ref[k])
        nm_ref[:, k, :] = r
        hm_ref[k] = r


def _combine(acc, ht, exself, den, adm, b):
    return pl.pallas_call(
        _combine_body,
        grid=(NB,),
        in_specs=[
            pl.BlockSpec((H, 1, BN, CP), lambda i: (0, 0, i, 0)),
            pl.BlockSpec((H, 1, BN, CP), lambda i: (0, 1, i, 0)),
            pl.BlockSpec((H, BN, CP), lambda i: (0, i, 0)),
            pl.BlockSpec((BN, H), lambda i: (i, 0)),
            pl.BlockSpec((2, BN, H), lambda i: (0, i, 0)),
            pl.BlockSpec((BN, 2 * H), lambda i: (i, 0)),
            pl.BlockSpec((H, 1, CP), lambda i: (0, 0, 0)),
        ],
        out_specs=[
            pl.BlockSpec((BN, H, CP), lambda i: (i, 0, 0)),
            pl.BlockSpec((H, BN, CP), lambda i: (0, i, 0)),
        ],
        out_shape=[
            jax.ShapeDtypeStruct((N, H, CP), jnp.float32),
            jax.ShapeDtypeStruct((H, N, CP), jnp.float32),
        ],
    )(acc, acc, ht, exself, den, adm, b)


def _graphconv_body(h2_ref, nbr_ref, wgr_ref, wgn_ref, bg_ref, o_ref):
    acc = jnp.dot(h2_ref[...], wgr_ref[...],
                  preferred_element_type=jnp.float32) + bg_ref[...]
    for k in range(H):
        acc += jnp.dot(nbr_ref[k], wgn_ref[k],
                       preferred_element_type=jnp.float32)
    o_ref[...] = jax.nn.relu(acc)


def _graphconv(h2n, nbrg, wgr, wgn, bg):
    return pl.pallas_call(
        _graphconv_body,
        grid=(NB,),
        in_specs=[
            pl.BlockSpec((BN, H * CP), lambda i: (i, 0)),
            pl.BlockSpec((H, BN, CP), lambda i: (0, i, 0)),
            pl.BlockSpec((H * CP, 48), lambda i: (0, 0)),
            pl.BlockSpec((H, CP, 48), lambda i: (0, 0, 0)),
            pl.BlockSpec((1, 48), lambda i: (0, 0)),
        ],
        out_specs=pl.BlockSpec((BN, 48), lambda i: (i, 0)),
        out_shape=jax.ShapeDtypeStruct((N, 48), jnp.float32),
    )(h2n, nbrg, wgr, wgn, bg)


def _head_body(h3_ref, nbr_ref, deg_ref, wsn_ref, wsr_ref, bs_ref,
               batch_ref, fc1w_ref, fc1b_ref, fc3w_ref, fc3b_ref,
               out_ref, sums_ref, cnt_ref):
    i = pl.program_id(0)

    @pl.when(i == 0)
    def _():
        sums_ref[...] = jnp.zeros_like(sums_ref)
        cnt_ref[...] = jnp.zeros_like(cnt_ref)

    nm = nbr_ref[...] / jnp.maximum(deg_ref[...], 1.0)
    h4 = jax.nn.relu(
        jnp.dot(nm, wsn_ref[...], preferred_element_type=jnp.float32)
        + jnp.dot(h3_ref[...], wsr_ref[...],
                  preferred_element_type=jnp.float32)
        + bs_ref[...])
    onehot = (batch_ref[...] == lax.broadcasted_iota(jnp.int32, (1, G), 1)
              ).astype(jnp.float32)
    sums_ref[...] += lax.dot_general(onehot, h4, (((0,), (0,)), ((), ())),
                                     preferred_element_type=jnp.float32)
    cnt_ref[...] += jnp.sum(onehot, axis=0, keepdims=True)

    @pl.when(i == NB - 1)
    def _():
        g = sums_ref[...] / jnp.maximum(cnt_ref[...], 1.0).T
        g = jax.nn.relu(g @ fc1w_ref[...] + fc1b_ref[...])
        out_ref[...] = g @ fc3w_ref[...] + fc3b_ref[...]


def _head(h3, nbr3, deg, wsn, wsr, bs, batch, fc1w, fc1b, fc3w, fc3b):
    return pl.pallas_call(
        _head_body,
        grid=(NB,),
        in_specs=[
            pl.BlockSpec((BN, 48), lambda i: (i, 0)),
            pl.BlockSpec((BN, 48), lambda i: (i, 0)),
            pl.BlockSpec((BN, 1), lambda i: (i, 0)),
            pl.BlockSpec((48, 48), lambda i: (0, 0)),
            pl.BlockSpec((48, 48), lambda i: (0, 0)),
            pl.BlockSpec((1, 48), lambda i: (0, 0)),
            pl.BlockSpec((BN, 1), lambda i: (i, 0)),
            pl.BlockSpec((48, 32), lambda i: (0, 0)),
            pl.BlockSpec((1, 32), lambda i: (0, 0)),
            pl.BlockSpec((32, 10), lambda i: (0, 0)),
            pl.BlockSpec((1, 10), lambda i: (0, 0)),
        ],
        out_specs=pl.BlockSpec((G, 10), lambda i: (0, 0)),
        out_shape=jax.ShapeDtypeStruct((G, 10), jnp.float32),
        scratch_shapes=[
            pltpu.VMEM((G, 48), jnp.float32),
            pltpu.VMEM((1, G), jnp.float32),
        ],
    )(h3, nbr3, deg, wsn, wsr, bs, batch, fc1w, fc1b, fc3w, fc3b)


def _sc_softmax_body(src_hbm, dst_hbm, as_hbm, adm_hbm, z_hbm,
                     den_hbm, gidx_hbm, sidx_hbm,
                     srcb, dstb, asb, admb, exb, gib, sib, den_sh):
    c = lax.axis_index("c")
    s = lax.axis_index("s")
    rpt = NP // NTILE
    r0 = s * rpt
    pltpu.sync_copy(z_hbm.at[pl.ds(r0, rpt)], den_sh.at[pl.ds(r0, rpt)])
    plsc.subcore_barrier()
    epc = E // 2
    ept = epc // NTILE

    @pl.loop(0, ept // CH)
    def _(ci):
        base = c * epc + s * ept + ci * CH
        pltpu.sync_copy(src_hbm.at[pl.ds(base, CH)], srcb)
        pltpu.sync_copy(dst_hbm.at[pl.ds(base, CH)], dstb)
        pltpu.sync_copy(as_hbm.at[srcb], asb)
        pltpu.sync_copy(adm_hbm.at[dstb], admb)

        @pl.loop(0, CH)
        def _(i):
            ei = asb[i, :] + admb[i, pl.ds(0, H)]
            e = jnp.maximum(ei, 0.2 * ei)
            exb[i, :] = jnp.exp(e - admb[i, pl.ds(H, H)])
            off = jnp.where(ei < 0.0, NP, 0).astype(jnp.int32)
            srcv = plsc.load_gather(srcb, [jnp.full((16,), i, jnp.int32)])
            dstv = plsc.load_gather(dstb, [jnp.full((16,), i, jnp.int32)])
            gib[i, :] = srcv + off
            sib[i, :] = dstv + off

        pltpu.sync_copy(exb, den_sh.at[dstb], add=True)
        pltpu.sync_copy(gib, gidx_hbm.at[pl.ds(base, CH)])
        pltpu.sync_copy(sib, sidx_hbm.at[pl.ds(base, CH)])

    plsc.subcore_barrier()

    @pl.when(c == 0)
    def _():
        pltpu.sync_copy(den_sh.at[pl.ds(r0, rpt)],
                        den_hbm.at[0, pl.ds(r0, rpt)])

    @pl.when(c == 1)
    def _():
        pltpu.sync_copy(den_sh.at[pl.ds(r0, rpt)],
                        den_hbm.at[1, pl.ds(r0, rpt)])


def _sc_softmax(src, dst, a_s, adm, z16):
    k = functools.partial(
        pl.kernel, mesh=_mesh, compiler_params=_sc_params,
        out_type=[jax.ShapeDtypeStruct((2, NP, H), jnp.float32),
                  jax.ShapeDtypeStruct((E, H), jnp.int32),
                  jax.ShapeDtypeStruct((E, H), jnp.int32)],
        scratch_types=[
            pltpu.VMEM((CH,), jnp.int32),
            pltpu.VMEM((CH,), jnp.int32),
            pltpu.VMEM((CH, H), jnp.float32),
            pltpu.VMEM((CH, 2 * H), jnp.float32),
            pltpu.VMEM((CH, H), jnp.float32),
            pltpu.VMEM((CH, H), jnp.int32),
            pltpu.VMEM((CH, H), jnp.int32),
            pltpu.VMEM_SHARED((NP, H), jnp.float32),
        ])(_sc_softmax_body)
    return k(src, dst, a_s, adm, z16)


def _make_stream_body(M, CHX, GRP, shared):
    nch = (E // NTILE) // CHX
    ngroups = nch // GRP

    def body(t_hbm, g3_hbm, s3_hbm, z_hbm, out_hbm,
             gib, sib, rows, acc_sh, sg0, sg1):
        c = lax.axis_index("c")
        s = lax.axis_index("s")
        rpt = M // NTILE
        r0 = s * rpt
        q0 = s * nch

        sems = (sg0, sg1)

        @pl.loop(0, H // 2)
        def _(kl):
            k = 2 * kl + c
            kk = 0 if shared else k
            tref = t_hbm.at[k]

            def load_group(slot, gi):
                pltpu.sync_copy(g3_hbm.at[kk, pl.ds(q0 + gi * GRP, GRP)],
                                gib.at[slot])
                pltpu.sync_copy(s3_hbm.at[kk, pl.ds(q0 + gi * GRP, GRP)],
                                sib.at[slot])

            def start_gather(buf, slot, row):
                pltpu.async_copy(tref.at[gib.at[slot, row]],
                                 rows.at[buf], sems[buf])

            def wait_gather(buf, slot):
                pltpu.make_async_copy(tref.at[gib.at[slot, 0]],
                                      rows.at[buf], sems[buf]).wait()

            def scatter(buf, slot, row):
                pltpu.sync_copy(rows.at[buf], acc_sh.at[sib.at[slot, row]],
                                add=True)

            pltpu.sync_copy(z_hbm.at[pl.ds(r0, rpt)],
                            acc_sh.at[pl.ds(r0, rpt)])
            plsc.subcore_barrier()

            load_group(0, 0)
            start_gather(0, 0, 0)
            for g in range(ngroups):
                slot = g % 2

                @pl.loop(0, GRP // 2 - 1)
                def _(jj):
                    r = 2 * jj
                    start_gather(1, slot, r + 1)
                    wait_gather(0, slot)
                    scatter(0, slot, r)
                    start_gather(0, slot, r + 2)
                    wait_gather(1, slot)
                    scatter(1, slot, r + 1)

                r = GRP - 2
                start_gather(1, slot, r + 1)
                wait_gather(0, slot)
                scatter(0, slot, r)
                if g + 1 < ngroups:
                    load_group(1 - slot, g + 1)
                    start_gather(0, 1 - slot, 0)
                wait_gather(1, slot)
                scatter(1, slot, r + 1)

            plsc.subcore_barrier()
            pltpu.sync_copy(acc_sh.at[pl.ds(r0, rpt)],
                            out_hbm.at[k, pl.ds(r0, rpt)])

    return body


CHB2 = 200
CHC2 = 400
GRPB = 20
GRPC = 10


def _sc_gat_agg(t2, gidx_t, sidx_t, z2):
    g3 = gidx_t.reshape(H, E // CHB2, CHB2)
    s3 = sidx_t.reshape(H, E // CHB2, CHB2)
    k = functools.partial(
        pl.kernel, mesh=_mesh, compiler_params=_sc_params,
        out_type=jax.ShapeDtypeStruct((H, 2 * NP, CP), jnp.float32),
        scratch_types=[
            pltpu.VMEM((2, GRPB, CHB2), jnp.int32),
            pltpu.VMEM((2, GRPB, CHB2), jnp.int32),
            pltpu.VMEM((2, CHB2, CP), jnp.float32),
            pltpu.VMEM_SHARED((2 * NP, CP), jnp.float32),
            pltpu.SemaphoreType.DMA,
            pltpu.SemaphoreType.DMA,
        ])(_make_stream_body(2 * NP, CHB2, GRPB, False))
    return k(t2, g3, s3, z2)


def _sc_sum_agg(ht, src_e, dst_e, z64):
    g3 = src_e.reshape(1, E // CHC2, CHC2)
    s3 = dst_e.reshape(1, E // CHC2, CHC2)
    k = functools.partial(
        pl.kernel, mesh=_mesh, compiler_params=_sc_params,
        out_type=jax.ShapeDtypeStruct((H, NP, CP), jnp.float32),
        scratch_types=[
            pltpu.VMEM((2, GRPC, CHC2), jnp.int32),
            pltpu.VMEM((2, GRPC, CHC2), jnp.int32),
            pltpu.VMEM((2, CHC2, CP), jnp.float32),
            pltpu.VMEM_SHARED((NP, CP), jnp.float32),
            pltpu.SemaphoreType.DMA,
            pltpu.SemaphoreType.DMA,
        ])(_make_stream_body(NP, CHC2, GRPC, True))
    return k(ht, g3, s3, z64)


def _sc_sage_body(h3_hbm, src_hbm, dst_hbm, ones_hbm, z48_hbm, z16_hbm,
                  nbr_hbm, deg_hbm, srcb, dstb, rows, onesb,
                  nbr_sh, deg_sh):
    c = lax.axis_index("c")
    s = lax.axis_index("s")
    rpt = NP // NTILE
    r0 = s * rpt
    ept = E // NTILE

    @pl.when(c == 0)
    def _():
        pltpu.sync_copy(z48_hbm.at[pl.ds(r0, rpt)],
                        nbr_sh.at[pl.ds(r0, rpt)])
        plsc.subcore_barrier()

        @pl.loop(0, ept // CH)
        def _(ci):
            b = s * ept + ci * CH
            pltpu.sync_copy(src_hbm.at[pl.ds(b, CH)], srcb)
            pltpu.sync_copy(dst_hbm.at[pl.ds(b, CH)], dstb)
            pltpu.sync_copy(h3_hbm.at[srcb], rows)
            pltpu.sync_copy(rows, nbr_sh.at[dstb], add=True)

        plsc.subcore_barrier()
        pltpu.sync_copy(nbr_sh.at[pl.ds(r0, rpt)],
                        nbr_hbm.at[pl.ds(r0, rpt)])

    @pl.when(c == 1)
    def _():
        pltpu.sync_copy(z16_hbm.at[pl.ds(r0, rpt)],
                        deg_sh.at[pl.ds(r0, rpt)])
        pltpu.sync_copy(ones_hbm, onesb)
        plsc.subcore_barrier()

        @pl.loop(0, ept // CH)
        def _(ci):
            b = s * ept + ci * CH
            pltpu.sync_copy(dst_hbm.at[pl.ds(b, CH)], dstb)
            pltpu.sync_copy(onesb, deg_sh.at[dstb], add=True)

        plsc.subcore_barrier()
        pltpu.sync_copy(deg_sh.at[pl.ds(r0, rpt)],
                        deg_hbm.at[pl.ds(r0, rpt)])


def _sc_sage(h3, src, dst, ones, z48, z16):
    k = functools.partial(
        pl.kernel, mesh=_mesh, compiler_params=_sc_params,
        out_type=[jax.ShapeDtypeStruct((NP, 48), jnp.float32),
                  jax.ShapeDtypeStruct((NP, H), jnp.float32)],
        scratch_types=[
            pltpu.VMEM((CH,), jnp.int32),
            pltpu.VMEM((CH,), jnp.int32),
            pltpu.VMEM((CH, 48), jnp.float32),
            pltpu.VMEM((CH, H), jnp.float32),
            pltpu.VMEM_SHARED((N, 48), jnp.float32),
            pltpu.VMEM_SHARED((NP, H), jnp.float32),
        ])(_sc_sage_body)
    return k(h3, src, dst, ones, z48, z16)


def _gat_layer(x, src, dst, wt, asr, adr, bp, z16, z2):
    ht = _mm(x, wt)
    a_s, a_d = _asad(ht, asr, adr)
    adm, exself = _prep(a_s, a_d)
    den, gidx, sidx = _sc_softmax(src, dst, a_s, adm, z16)
    gidx_t = _tr(gidx)
    sidx_t = _tr(sidx)
    t2 = _scale_tables(ht, a_s).reshape(H, 2 * NP, CP)
    acc = _sc_gat_agg(t2, gidx_t, sidx_t, z2).reshape(H, 2, NP, CP)
    return _combine(acc, ht, exself, den, adm, bp)


def kernel(x, edge_index, batch, W1, att_src1, att_dst1, b1, W2, att_src2,
           att_dst2, b2, Wg_root, Wg_nbr, bg, Ws_nbr, Ws_root, bs, fc1_w,
           fc1_b, fc3_w, fc3_b):
    f32 = jnp.float32
    src = edge_index[0]
    dst = edge_index[1]
    C1 = att_src1.shape[1]
    C2 = att_src2.shape[1]

    def padh(a, c):
        return jnp.pad(a.reshape(H, 1, c), ((0, 0), (0, 0), (0, CP - c)))

    w1t = jnp.pad(W1.reshape(-1, H, C1), ((0, 0), (0, 0), (0, CP - C1))
                  ).transpose(1, 0, 2)
    w2t = jnp.pad(W2.reshape(H, C1, H, C2),
                  ((0, 0), (0, CP - C1), (0, 0), (0, CP - C2))
                  ).reshape(H * CP, H, CP).transpose(1, 0, 2)
    wgr = jnp.pad(Wg_root.reshape(H, C2, 40),
                  ((0, 0), (0, CP - C2), (0, 8))).reshape(H * CP, 48)
    wgn = jnp.pad(Wg_nbr.reshape(H, C2, 40),
                  ((0, 0), (0, CP - C2), (0, 8)))
    wsn = jnp.pad(Ws_nbr, ((0, 8), (0, 8)))
    wsr = jnp.pad(Ws_root, ((0, 8), (0, 8)))
    fc1p = jnp.pad(fc1_w, ((0, 8), (0, 0)))

    z16 = jnp.zeros((NP, H), f32)
    z2 = jnp.zeros((2 * NP, CP), f32)
    z48 = jnp.zeros((NP, 48), f32)
    z64 = jnp.zeros((NP, CP), f32)
    ones = jnp.ones((CH, H), f32)

    h2nm, _ = _gat_layer(x, src, dst, w1t, padh(att_src1, C1),
                         padh(att_dst1, C1), padh(b1, C1), z16, z2)
    h2 = h2nm.reshape(N, H * CP)
    h3nm, h3hm = _gat_layer(h2, src, dst, w2t, padh(att_src2, C2),
                            padh(att_dst2, C2), padh(b2, C2), z16, z2)
    h3n = h3nm.reshape(N, H * CP)

    nbrg = _sc_sum_agg(h3hm, src, dst, z64)
    h4 = _graphconv(h3n, nbrg, wgr, wgn, jnp.pad(bg, (0, 8)).reshape(1, 48))

    nbr3, deg = _sc_sage(h4, src, dst, ones, z48, z16)
    out = _head(h4, nbr3, deg[:, :1], wsn, wsr,
                jnp.pad(bs, (0, 8)).reshape(1, 48), batch.reshape(N, 1),
                fc1p, fc1_b.reshape(1, 32), fc3_w, fc3_b.reshape(1, 10))
    return out


# speedup vs baseline: 17.7681x; 1.0146x over previous
import functools

import jax
import jax.numpy as jnp
from jax import lax
from jax.experimental import pallas as pl
from jax.experimental.pallas import tpu as pltpu
from jax.experimental.pallas import tpu_sc as plsc

N = 10000
E = 320000
G = 64
H = 16
CP = 64
NP = 10240
BN = 512
NB = NP // BN
NTILE = 16
CH = 1000
CHB = 400
_mesh = plsc.VectorSubcoreMesh(core_axis_name="c", subcore_axis_name="s")
_sc_params = pltpu.CompilerParams(use_tc_tiling_on_sc=False, needs_layout_passes=False)


def _mm_body(x_ref, w_ref, ht_ref):
    ht_ref[0] = jnp.dot(x_ref[...], w_ref[0],
                        preferred_element_type=jnp.float32)


def _mm(x, wt):
    d = x.shape[1]
    return pl.pallas_call(
        _mm_body,
        grid=(NB, H),
        in_specs=[
            pl.BlockSpec((BN, d), lambda i, k: (i, 0)),
            pl.BlockSpec((1, d, CP), lambda i, k: (k, 0, 0)),
        ],
        out_specs=pl.BlockSpec((1, BN, CP), lambda i, k: (k, i, 0)),
        out_shape=jax.ShapeDtypeStruct((H, NP, CP), jnp.float32),
    )(x, wt)


def _asad_body(ht_ref, asr_ref, adr_ref, as_ref, ad_ref):
    cols_s = []
    cols_d = []
    for k in range(H):
        h = ht_ref[k]
        cols_s.append(jnp.sum(h * asr_ref[k], axis=1, keepdims=True))
        cols_d.append(jnp.sum(h * adr_ref[k], axis=1, keepdims=True))
    as_ref[...] = jnp.concatenate(cols_s, axis=1)
    ad_ref[...] = jnp.concatenate(cols_d, axis=1)


def _asad(ht, asr, adr):
    return pl.pallas_call(
        _asad_body,
        grid=(NB,),
        in_specs=[
            pl.BlockSpec((H, BN, CP), lambda i: (0, i, 0)),
            pl.BlockSpec((H, 1, CP), lambda i: (0, 0, 0)),
            pl.BlockSpec((H, 1, CP), lambda i: (0, 0, 0)),
        ],
        out_specs=[
            pl.BlockSpec((BN, H), lambda i: (i, 0)),
            pl.BlockSpec((BN, H), lambda i: (i, 0)),
        ],
        out_shape=[
            jax.ShapeDtypeStruct((NP, H), jnp.float32),
            jax.ShapeDtypeStruct((NP, H), jnp.float32),
        ],
    )(ht, asr, adr)


def _prep_body(as_ref, ad_ref, adm_ref, exself_ref):
    a_s = as_ref[...]
    a_d = ad_ref[...]
    mb = jnp.max(a_s, axis=0, keepdims=True) + a_d
    m = jnp.maximum(mb, 0.2 * mb)
    adm_ref[:, :H] = a_d
    adm_ref[:, H:] = m
    e = a_s + a_d
    e = jnp.maximum(e, 0.2 * e)
    exself_ref[...] = jnp.exp(e - m)


def _prep(a_s, a_d):
    return pl.pallas_call(
        _prep_body,
        out_shape=[
            jax.ShapeDtypeStruct((NP, 2 * H), jnp.float32),
            jax.ShapeDtypeStruct((NP, H), jnp.float32),
        ],
    )(a_s, a_d)


def _tr_body(x_ref, o_ref):
    o_ref[...] = x_ref[...].T


def _tr(ex):
    be = 6400
    return pl.pallas_call(
        _tr_body,
        grid=(E // be,),
        in_specs=[pl.BlockSpec((be, H), lambda i: (i, 0))],
        out_specs=pl.BlockSpec((H, be), lambda i: (0, i)),
        out_shape=jax.ShapeDtypeStruct((H, E), ex.dtype),
    )(ex).reshape(H, E // CHB2, CHB2)


def _scale_body(ht_ref, as_ref, t2_ref):
    a_s = as_ref[...]
    for k in range(H):
        h = ht_ref[k]
        t2_ref[k, 0] = h * jnp.exp(a_s[:, k:k + 1])
        t2_ref[k, 1] = h * jnp.exp(0.2 * a_s[:, k:k + 1])


def _scale_tables(ht, a_s):
    return pl.pallas_call(
        _scale_body,
        grid=(NB,),
        in_specs=[
            pl.BlockSpec((H, BN, CP), lambda i: (0, i, 0)),
            pl.BlockSpec((BN, H), lambda i: (i, 0)),
        ],
        out_specs=pl.BlockSpec((H, 2, BN, CP), lambda i: (0, 0, i, 0)),
        out_shape=jax.ShapeDtypeStruct((H, 2, NP, CP), jnp.float32),
    )(ht, a_s)


def _combine_body(accp_ref, accn_ref, ht_ref, exself_ref, den_ref,
                  adm_ref, b_ref, nm_ref, hm_ref):
    exs = exself_ref[...]
    adm = adm_ref[...]
    den = den_ref[0] + den_ref[1] + exs + 1e-16
    for k in range(H):
        ad_k = adm[:, k:k + 1]
        m_k = adm[:, H + k:H + k + 1]
        num = (jnp.exp(ad_k - m_k) * accp_ref[k, 0]
               + jnp.exp(0.2 * ad_k - m_k) * accn_ref[k, 0]
               + exs[:, k:k + 1] * ht_ref[k])
        r = jax.nn.relu(num / den[:, k:k + 1] + b_ref[k])
        nm_ref[:, k, :] = r
        hm_ref[k] = r


def _combine(acc, ht, exself, den, adm, b):
    return pl.pallas_call(
        _combine_body,
        grid=(NB,),
        in_specs=[
            pl.BlockSpec((H, 1, BN, CP), lambda i: (0, 0, i, 0)),
            pl.BlockSpec((H, 1, BN, CP), lambda i: (0, 1, i, 0)),
            pl.BlockSpec((H, BN, CP), lambda i: (0, i, 0)),
            pl.BlockSpec((BN, H), lambda i: (i, 0)),
            pl.BlockSpec((2, BN, H), lambda i: (0, i, 0)),
            pl.BlockSpec((BN, 2 * H), lambda i: (i, 0)),
            pl.BlockSpec((H, 1, CP), lambda i: (0, 0, 0)),
        ],
        out_specs=[
            pl.BlockSpec((BN, H, CP), lambda i: (i, 0, 0)),
            pl.BlockSpec((H, BN, CP), lambda i: (0, i, 0)),
        ],
        out_shape=[
            jax.ShapeDtypeStruct((NP, H, CP), jnp.float32),
            jax.ShapeDtypeStruct((H, NP, CP), jnp.float32),
        ],
    )(acc, acc, ht, exself, den, adm, b)


def _graphconv_body(h2_ref, nbr_ref, wgr_ref, wgn_ref, bg_ref, o_ref):
    acc = jnp.dot(h2_ref[...], wgr_ref[...],
                  preferred_element_type=jnp.float32) + bg_ref[...]
    for k in range(H):
        acc += jnp.dot(nbr_ref[k], wgn_ref[k],
                       preferred_element_type=jnp.float32)
    o_ref[...] = jax.nn.relu(acc)


def _graphconv(h2n, nbrg, wgr, wgn, bg):
    return pl.pallas_call(
        _graphconv_body,
        grid=(NB,),
        in_specs=[
            pl.BlockSpec((BN, H * CP), lambda i: (i, 0)),
            pl.BlockSpec((H, BN, CP), lambda i: (0, i, 0)),
            pl.BlockSpec((H * CP, 48), lambda i: (0, 0)),
            pl.BlockSpec((H, CP, 48), lambda i: (0, 0, 0)),
            pl.BlockSpec((1, 48), lambda i: (0, 0)),
        ],
        out_specs=pl.BlockSpec((BN, 48), lambda i: (i, 0)),
        out_shape=jax.ShapeDtypeStruct((NP, 48), jnp.float32),
    )(h2n, nbrg, wgr, wgn, bg)


def _head_body(h3_ref, nbr_ref, deg_ref, wsn_ref, wsr_ref, bs_ref,
               batch_ref, fc1w_ref, fc1b_ref, fc3w_ref, fc3b_ref,
               out_ref, sums_ref, cnt_ref):
    i = pl.program_id(0)

    @pl.when(i == 0)
    def _():
        sums_ref[...] = jnp.zeros_like(sums_ref)
        cnt_ref[...] = jnp.zeros_like(cnt_ref)

    nm = nbr_ref[...] / jnp.maximum(deg_ref[...], 1.0)
    h4 = jax.nn.relu(
        jnp.dot(nm, wsn_ref[...], preferred_element_type=jnp.float32)
        + jnp.dot(h3_ref[...], wsr_ref[...],
                  preferred_element_type=jnp.float32)
        + bs_ref[...])
    onehot = (batch_ref[...] == lax.broadcasted_iota(jnp.int32, (1, G), 1)
              ).astype(jnp.float32)
    sums_ref[...] += lax.dot_general(onehot, h4, (((0,), (0,)), ((), ())),
                                     preferred_element_type=jnp.float32)
    cnt_ref[...] += jnp.sum(onehot, axis=0, keepdims=True)

    @pl.when(i == NB - 1)
    def _():
        g = sums_ref[...] / jnp.maximum(cnt_ref[...], 1.0).T
        g = jax.nn.relu(g @ fc1w_ref[...] + fc1b_ref[...])
        out_ref[...] = g @ fc3w_ref[...] + fc3b_ref[...]


def _head(h3, nbr3, deg, wsn, wsr, bs, batch, fc1w, fc1b, fc3w, fc3b):
    return pl.pallas_call(
        _head_body,
        grid=(NB,),
        in_specs=[
            pl.BlockSpec((BN, 48), lambda i: (i, 0)),
            pl.BlockSpec((BN, 48), lambda i: (i, 0)),
            pl.BlockSpec((BN, 1), lambda i: (i, 0)),
            pl.BlockSpec((48, 48), lambda i: (0, 0)),
            pl.BlockSpec((48, 48), lambda i: (0, 0)),
            pl.BlockSpec((1, 48), lambda i: (0, 0)),
            pl.BlockSpec((BN, 1), lambda i: (i, 0)),
            pl.BlockSpec((48, 32), lambda i: (0, 0)),
            pl.BlockSpec((1, 32), lambda i: (0, 0)),
            pl.BlockSpec((32, 10), lambda i: (0, 0)),
            pl.BlockSpec((1, 10), lambda i: (0, 0)),
        ],
        out_specs=pl.BlockSpec((G, 10), lambda i: (0, 0)),
        out_shape=jax.ShapeDtypeStruct((G, 10), jnp.float32),
        scratch_shapes=[
            pltpu.VMEM((G, 48), jnp.float32),
            pltpu.VMEM((1, G), jnp.float32),
        ],
    )(h3, nbr3, deg, wsn, wsr, bs, batch, fc1w, fc1b, fc3w, fc3b)


def _sc_softmax_body(src_hbm, dst_hbm, as_hbm, adm_hbm, z_hbm,
                     den_hbm, gidx_hbm, sidx_hbm,
                     srcb, dstb, asb, admb, exb, gib, sib, den_sh):
    c = lax.axis_index("c")
    s = lax.axis_index("s")
    rpt = NP // NTILE
    r0 = s * rpt
    pltpu.sync_copy(z_hbm.at[pl.ds(r0, rpt)], den_sh.at[pl.ds(r0, rpt)])
    plsc.subcore_barrier()
    epc = E // 2
    ept = epc // NTILE

    @pl.loop(0, ept // CH)
    def _(ci):
        base = c * epc + s * ept + ci * CH
        pltpu.sync_copy(src_hbm.at[pl.ds(base, CH)], srcb)
        pltpu.sync_copy(dst_hbm.at[pl.ds(base, CH)], dstb)
        pltpu.sync_copy(as_hbm.at[srcb], asb)
        pltpu.sync_copy(adm_hbm.at[dstb], admb)

        @pl.loop(0, CH)
        def _(i):
            ei = asb[i, :] + admb[i, pl.ds(0, H)]
            e = jnp.maximum(ei, 0.2 * ei)
            exb[i, :] = jnp.exp(e - admb[i, pl.ds(H, H)])
            off = jnp.where(ei < 0.0, NP, 0).astype(jnp.int32)
            srcv = plsc.load_gather(srcb, [jnp.full((16,), i, jnp.int32)])
            dstv = plsc.load_gather(dstb, [jnp.full((16,), i, jnp.int32)])
            gib[i, :] = srcv + off
            sib[i, :] = dstv + off

        pltpu.sync_copy(exb, den_sh.at[dstb], add=True)
        pltpu.sync_copy(gib, gidx_hbm.at[pl.ds(base, CH)])
        pltpu.sync_copy(sib, sidx_hbm.at[pl.ds(base, CH)])

    plsc.subcore_barrier()

    @pl.when(c == 0)
    def _():
        pltpu.sync_copy(den_sh.at[pl.ds(r0, rpt)],
                        den_hbm.at[0, pl.ds(r0, rpt)])

    @pl.when(c == 1)
    def _():
        pltpu.sync_copy(den_sh.at[pl.ds(r0, rpt)],
                        den_hbm.at[1, pl.ds(r0, rpt)])


def _sc_softmax(src, dst, a_s, adm, z16):
    k = functools.partial(
        pl.kernel, mesh=_mesh, compiler_params=_sc_params,
        out_type=[jax.ShapeDtypeStruct((2, NP, H), jnp.float32),
                  jax.ShapeDtypeStruct((E, H), jnp.int32),
                  jax.ShapeDtypeStruct((E, H), jnp.int32)],
        scratch_types=[
            pltpu.VMEM((CH,), jnp.int32),
            pltpu.VMEM((CH,), jnp.int32),
            pltpu.VMEM((CH, H), jnp.float32),
            pltpu.VMEM((CH, 2 * H), jnp.float32),
            pltpu.VMEM((CH, H), jnp.float32),
            pltpu.VMEM((CH, H), jnp.int32),
            pltpu.VMEM((CH, H), jnp.int32),
            pltpu.VMEM_SHARED((NP, H), jnp.float32),
        ])(_sc_softmax_body)
    return k(src, dst, a_s, adm, z16)


def _make_stream_body(M, CHX, GRP, shared):
    nch = (E // NTILE) // CHX
    ngroups = nch // GRP

    def body(t_hbm, g3_hbm, s3_hbm, z_hbm, out_hbm,
             gib, sib, rows, acc_sh, sg0, sg1):
        c = lax.axis_index("c")
        s = lax.axis_index("s")
        rpt = M // NTILE
        r0 = s * rpt
        q0 = s * nch

        sems = (sg0, sg1)

        @pl.loop(0, H // 2)
        def _(kl):
            k = 2 * kl + c
            kk = 0 if shared else k
            tref = t_hbm.at[k]

            def load_group(slot, gi):
                pltpu.sync_copy(g3_hbm.at[kk, pl.ds(q0 + gi * GRP, GRP)],
                                gib.at[slot])
                pltpu.sync_copy(s3_hbm.at[kk, pl.ds(q0 + gi * GRP, GRP)],
                                sib.at[slot])

            def start_gather(buf, slot, row):
                pltpu.async_copy(tref.at[gib.at[slot, row]],
                                 rows.at[buf], sems[buf])

            def wait_gather(buf, slot):
                pltpu.make_async_copy(tref.at[gib.at[slot, 0]],
                                      rows.at[buf], sems[buf]).wait()

            def scatter(buf, slot, row):
                pltpu.sync_copy(rows.at[buf], acc_sh.at[sib.at[slot, row]],
                                add=True)

            pltpu.sync_copy(z_hbm.at[pl.ds(r0, rpt)],
                            acc_sh.at[pl.ds(r0, rpt)])
            plsc.subcore_barrier()

            load_group(0, 0)
            start_gather(0, 0, 0)
            for g in range(ngroups):
                slot = g % 2

                @pl.loop(0, GRP // 2 - 1)
                def _(jj):
                    r = 2 * jj
                    start_gather(1, slot, r + 1)
                    wait_gather(0, slot)
                    scatter(0, slot, r)
                    start_gather(0, slot, r + 2)
                    wait_gather(1, slot)
                    scatter(1, slot, r + 1)

                r = GRP - 2
                start_gather(1, slot, r + 1)
                wait_gather(0, slot)
                scatter(0, slot, r)
                if g + 1 < ngroups:
                    load_group(1 - slot, g + 1)
                    start_gather(0, 1 - slot, 0)
                wait_gather(1, slot)
                scatter(1, slot, r + 1)

            plsc.subcore_barrier()
            pltpu.sync_copy(acc_sh.at[pl.ds(r0, rpt)],
                            out_hbm.at[k, pl.ds(r0, rpt)])

    return body


CHB2 = 200
CHC2 = 400
GRPB = 20
GRPC = 10


def _sc_gat_agg(t2, g3, s3, z2):
    k = functools.partial(
        pl.kernel, mesh=_mesh, compiler_params=_sc_params,
        out_type=jax.ShapeDtypeStruct((H, 2 * NP, CP), jnp.float32),
        scratch_types=[
            pltpu.VMEM((2, GRPB, CHB2), jnp.int32),
            pltpu.VMEM((2, GRPB, CHB2), jnp.int32),
            pltpu.VMEM((2, CHB2, CP), jnp.float32),
            pltpu.VMEM_SHARED((2 * NP, CP), jnp.float32),
            pltpu.SemaphoreType.DMA,
            pltpu.SemaphoreType.DMA,
        ])(_make_stream_body(2 * NP, CHB2, GRPB, False))
    return k(t2, g3, s3, z2)


def _sc_sum_agg(ht, src_e, dst_e, z64):
    g3 = src_e.reshape(1, E // CHC2, CHC2)
    s3 = dst_e.reshape(1, E // CHC2, CHC2)
    k = functools.partial(
        pl.kernel, mesh=_mesh, compiler_params=_sc_params,
        out_type=jax.ShapeDtypeStruct((H, NP, CP), jnp.float32),
        scratch_types=[
            pltpu.VMEM((2, GRPC, CHC2), jnp.int32),
            pltpu.VMEM((2, GRPC, CHC2), jnp.int32),
            pltpu.VMEM((2, CHC2, CP), jnp.float32),
            pltpu.VMEM_SHARED((NP, CP), jnp.float32),
            pltpu.SemaphoreType.DMA,
            pltpu.SemaphoreType.DMA,
        ])(_make_stream_body(NP, CHC2, GRPC, True))
    return k(ht, g3, s3, z64)


def _sc_sage_body(h3_hbm, src_hbm, dst_hbm, ones_hbm, z48_hbm, z16_hbm,
                  nbr_hbm, deg_hbm, srcb, dstb, rows, onesb,
                  nbr_sh, deg_sh):
    c = lax.axis_index("c")
    s = lax.axis_index("s")
    rpt = NP // NTILE
    r0 = s * rpt
    ept = E // NTILE

    @pl.when(c == 0)
    def _():
        pltpu.sync_copy(z48_hbm.at[pl.ds(r0, rpt)],
                        nbr_sh.at[pl.ds(r0, rpt)])
        plsc.subcore_barrier()

        @pl.loop(0, ept // CH)
        def _(ci):
            b = s * ept + ci * CH
            pltpu.sync_copy(src_hbm.at[pl.ds(b, CH)], srcb)
            pltpu.sync_copy(dst_hbm.at[pl.ds(b, CH)], dstb)
            pltpu.sync_copy(h3_hbm.at[srcb], rows)
            pltpu.sync_copy(rows, nbr_sh.at[dstb], add=True)

        plsc.subcore_barrier()
        pltpu.sync_copy(nbr_sh.at[pl.ds(r0, rpt)],
                        nbr_hbm.at[pl.ds(r0, rpt)])

    @pl.when(c == 1)
    def _():
        pltpu.sync_copy(z16_hbm.at[pl.ds(r0, rpt)],
                        deg_sh.at[pl.ds(r0, rpt)])
        pltpu.sync_copy(ones_hbm, onesb)
        plsc.subcore_barrier()

        @pl.loop(0, ept // CH)
        def _(ci):
            b = s * ept + ci * CH
            pltpu.sync_copy(dst_hbm.at[pl.ds(b, CH)], dstb)
            pltpu.sync_copy(onesb, deg_sh.at[dstb], add=True)

        plsc.subcore_barrier()
        pltpu.sync_copy(deg_sh.at[pl.ds(r0, rpt)],
                        deg_hbm.at[pl.ds(r0, rpt)])


def _sc_sage(h3, src, dst, ones, z48, z16):
    k = functools.partial(
        pl.kernel, mesh=_mesh, compiler_params=_sc_params,
        out_type=[jax.ShapeDtypeStruct((NP, 48), jnp.float32),
                  jax.ShapeDtypeStruct((NP, H), jnp.float32)],
        scratch_types=[
            pltpu.VMEM((CH,), jnp.int32),
            pltpu.VMEM((CH,), jnp.int32),
            pltpu.VMEM((CH, 48), jnp.float32),
            pltpu.VMEM((CH, H), jnp.float32),
            pltpu.VMEM_SHARED((N, 48), jnp.float32),
            pltpu.VMEM_SHARED((NP, H), jnp.float32),
        ])(_sc_sage_body)
    return k(h3, src, dst, ones, z48, z16)


def _gat_layer(x, src, dst, wt, asr, adr, bp, z16, z2):
    ht = _mm(x, wt)
    a_s, a_d = _asad(ht, asr, adr)
    adm, exself = _prep(a_s, a_d)
    den, gidx, sidx = _sc_softmax(src, dst, a_s, adm, z16)
    gidx_t = _tr(gidx)
    sidx_t = _tr(sidx)
    t2 = _scale_tables(ht, a_s).reshape(H, 2 * NP, CP)
    acc = _sc_gat_agg(t2, gidx_t, sidx_t, z2).reshape(H, 2, NP, CP)
    return _combine(acc, ht, exself, den, adm, bp)


def kernel(x, edge_index, batch, W1, att_src1, att_dst1, b1, W2, att_src2,
           att_dst2, b2, Wg_root, Wg_nbr, bg, Ws_nbr, Ws_root, bs, fc1_w,
           fc1_b, fc3_w, fc3_b):
    f32 = jnp.float32
    src = edge_index[0]
    dst = edge_index[1]
    C1 = att_src1.shape[1]
    C2 = att_src2.shape[1]

    def padh(a, c):
        return jnp.pad(a.reshape(H, 1, c), ((0, 0), (0, 0), (0, CP - c)))

    w1t = jnp.pad(W1.reshape(-1, H, C1), ((0, 0), (0, 0), (0, CP - C1))
                  ).transpose(1, 0, 2)
    w2t = jnp.pad(W2.reshape(H, C1, H, C2),
                  ((0, 0), (0, CP - C1), (0, 0), (0, CP - C2))
                  ).reshape(H * CP, H, CP).transpose(1, 0, 2)
    wgr = jnp.pad(Wg_root.reshape(H, C2, 40),
                  ((0, 0), (0, CP - C2), (0, 8))).reshape(H * CP, 48)
    wgn = jnp.pad(Wg_nbr.reshape(H, C2, 40),
                  ((0, 0), (0, CP - C2), (0, 8)))
    wsn = jnp.pad(Ws_nbr, ((0, 8), (0, 8)))
    wsr = jnp.pad(Ws_root, ((0, 8), (0, 8)))
    fc1p = jnp.pad(fc1_w, ((0, 8), (0, 0)))

    z16 = jnp.zeros((NP, H), f32)
    z2 = jnp.zeros((2 * NP, CP), f32)
    z48 = jnp.zeros((NP, 48), f32)
    z64 = jnp.zeros((NP, CP), f32)
    ones = jnp.ones((CH, H), f32)

    x = jnp.pad(x, ((0, NP - N), (0, 0)))
    batch = jnp.pad(batch, (0, NP - N), constant_values=G)

    h2nm, _ = _gat_layer(x, src, dst, w1t, padh(att_src1, C1),
                         padh(att_dst1, C1), padh(b1, C1), z16, z2)
    h2 = h2nm.reshape(NP, H * CP)
    h3nm, h3hm = _gat_layer(h2, src, dst, w2t, padh(att_src2, C2),
                            padh(att_dst2, C2), padh(b2, C2), z16, z2)
    h3n = h3nm.reshape(NP, H * CP)

    nbrg = _sc_sum_agg(h3hm, src, dst, z64)
    h4 = _graphconv(h3n, nbrg, wgr, wgn, jnp.pad(bg, (0, 8)).reshape(1, 48))

    nbr3, deg = _sc_sage(h4, src, dst, ones, z48, z16)
    out = _head(h4, nbr3, deg[:, :1], wsn, wsr,
                jnp.pad(bs, (0, 8)).reshape(1, 48), batch.reshape(NP, 1),
                fc1p, fc1_b.reshape(1, 32), fc3_w, fc3_b.reshape(1, 10))
    return out


# speedup vs baseline: 18.8915x; 1.0632x over previous
import functools

import jax
import jax.numpy as jnp
from jax import lax
from jax.experimental import pallas as pl
from jax.experimental.pallas import tpu as pltpu
from jax.experimental.pallas import tpu_sc as plsc

N = 10000
E = 320000
G = 64
H = 16
CP = 64
CP2 = 48
NP = 10240
BN = 512
NB = NP // BN
NTILE = 16
CH = 1000
CHB = 400
_mesh = plsc.VectorSubcoreMesh(core_axis_name="c", subcore_axis_name="s")
_sc_params = pltpu.CompilerParams(use_tc_tiling_on_sc=False, needs_layout_passes=False)


def _mm_body(x_ref, w_ref, ht_ref):
    ht_ref[0] = jnp.dot(x_ref[...], w_ref[0],
                        preferred_element_type=jnp.float32)


def _mm(x, wt):
    d = x.shape[1]
    cp = wt.shape[2]
    return pl.pallas_call(
        _mm_body,
        grid=(NB, H),
        in_specs=[
            pl.BlockSpec((BN, d), lambda i, k: (i, 0)),
            pl.BlockSpec((1, d, cp), lambda i, k: (k, 0, 0)),
        ],
        out_specs=pl.BlockSpec((1, BN, cp), lambda i, k: (k, i, 0)),
        out_shape=jax.ShapeDtypeStruct((H, NP, cp), jnp.float32),
    )(x, wt)


def _asad_body(ht_ref, asr_ref, adr_ref, as_ref, ad_ref):
    cols_s = []
    cols_d = []
    for k in range(H):
        h = ht_ref[k]
        cols_s.append(jnp.sum(h * asr_ref[k], axis=1, keepdims=True))
        cols_d.append(jnp.sum(h * adr_ref[k], axis=1, keepdims=True))
    as_ref[...] = jnp.concatenate(cols_s, axis=1)
    ad_ref[...] = jnp.concatenate(cols_d, axis=1)


def _asad(ht, asr, adr):
    cp = ht.shape[2]
    return pl.pallas_call(
        _asad_body,
        grid=(NB,),
        in_specs=[
            pl.BlockSpec((H, BN, cp), lambda i: (0, i, 0)),
            pl.BlockSpec((H, 1, cp), lambda i: (0, 0, 0)),
            pl.BlockSpec((H, 1, cp), lambda i: (0, 0, 0)),
        ],
        out_specs=[
            pl.BlockSpec((BN, H), lambda i: (i, 0)),
            pl.BlockSpec((BN, H), lambda i: (i, 0)),
        ],
        out_shape=[
            jax.ShapeDtypeStruct((NP, H), jnp.float32),
            jax.ShapeDtypeStruct((NP, H), jnp.float32),
        ],
    )(ht, asr, adr)


def _prep_body(as_ref, ad_ref, adm_ref, exself_ref):
    a_s = as_ref[...]
    a_d = ad_ref[...]
    mb = jnp.max(a_s, axis=0, keepdims=True) + a_d
    m = jnp.maximum(mb, 0.2 * mb)
    adm_ref[:, :H] = a_d
    adm_ref[:, H:] = m
    e = a_s + a_d
    e = jnp.maximum(e, 0.2 * e)
    exself_ref[...] = jnp.exp(e - m)


def _prep(a_s, a_d):
    return pl.pallas_call(
        _prep_body,
        out_shape=[
            jax.ShapeDtypeStruct((NP, 2 * H), jnp.float32),
            jax.ShapeDtypeStruct((NP, H), jnp.float32),
        ],
    )(a_s, a_d)


def _tr_body(x_ref, o_ref):
    o_ref[...] = x_ref[...].T


def _tr(ex):
    be = 6400
    return pl.pallas_call(
        _tr_body,
        grid=(E // be,),
        in_specs=[pl.BlockSpec((be, H), lambda i: (i, 0))],
        out_specs=pl.BlockSpec((H, be), lambda i: (0, i)),
        out_shape=jax.ShapeDtypeStruct((H, E), ex.dtype),
    )(ex).reshape(H, E // CHB2, CHB2)


def _scale_body(ht_ref, as_ref, t2_ref):
    a_s = as_ref[...]
    for k in range(H):
        h = ht_ref[k]
        t2_ref[k, 0] = h * jnp.exp(a_s[:, k:k + 1])
        t2_ref[k, 1] = h * jnp.exp(0.2 * a_s[:, k:k + 1])


def _scale_tables(ht, a_s):
    cp = ht.shape[2]
    return pl.pallas_call(
        _scale_body,
        grid=(NB,),
        in_specs=[
            pl.BlockSpec((H, BN, cp), lambda i: (0, i, 0)),
            pl.BlockSpec((BN, H), lambda i: (i, 0)),
        ],
        out_specs=pl.BlockSpec((H, 2, BN, cp), lambda i: (0, 0, i, 0)),
        out_shape=jax.ShapeDtypeStruct((H, 2, NP, cp), jnp.float32),
    )(ht, a_s)


def _combine_body(accp_ref, accn_ref, ht_ref, exself_ref, den_ref,
                  adm_ref, b_ref, nm_ref, hm_ref):
    exs = exself_ref[...]
    adm = adm_ref[...]
    den = den_ref[0] + den_ref[1] + exs + 1e-16
    for k in range(H):
        ad_k = adm[:, k:k + 1]
        m_k = adm[:, H + k:H + k + 1]
        num = (jnp.exp(ad_k - m_k) * accp_ref[k, 0]
               + jnp.exp(0.2 * ad_k - m_k) * accn_ref[k, 0]
               + exs[:, k:k + 1] * ht_ref[k])
        r = jax.nn.relu(num / den[:, k:k + 1] + b_ref[k])
        nm_ref[:, k, :] = r
        hm_ref[k] = r


def _combine(acc, ht, exself, den, adm, b):
    cp = ht.shape[2]
    return pl.pallas_call(
        _combine_body,
        grid=(NB,),
        in_specs=[
            pl.BlockSpec((H, 1, BN, cp), lambda i: (0, 0, i, 0)),
            pl.BlockSpec((H, 1, BN, cp), lambda i: (0, 1, i, 0)),
            pl.BlockSpec((H, BN, cp), lambda i: (0, i, 0)),
            pl.BlockSpec((BN, H), lambda i: (i, 0)),
            pl.BlockSpec((2, BN, H), lambda i: (0, i, 0)),
            pl.BlockSpec((BN, 2 * H), lambda i: (i, 0)),
            pl.BlockSpec((H, 1, cp), lambda i: (0, 0, 0)),
        ],
        out_specs=[
            pl.BlockSpec((BN, H, cp), lambda i: (i, 0, 0)),
            pl.BlockSpec((H, BN, cp), lambda i: (0, i, 0)),
        ],
        out_shape=[
            jax.ShapeDtypeStruct((NP, H, cp), jnp.float32),
            jax.ShapeDtypeStruct((H, NP, cp), jnp.float32),
        ],
    )(acc, acc, ht, exself, den, adm, b)


def _graphconv_body(h2_ref, nbr_ref, wgr_ref, wgn_ref, bg_ref, o_ref):
    acc = jnp.dot(h2_ref[...], wgr_ref[...],
                  preferred_element_type=jnp.float32) + bg_ref[...]
    for k in range(H):
        acc += jnp.dot(nbr_ref[k], wgn_ref[k],
                       preferred_element_type=jnp.float32)
    o_ref[...] = jax.nn.relu(acc)


def _graphconv(h2n, nbrg, wgr, wgn, bg):
    cp = nbrg.shape[2]
    return pl.pallas_call(
        _graphconv_body,
        grid=(NB,),
        in_specs=[
            pl.BlockSpec((BN, H * cp), lambda i: (i, 0)),
            pl.BlockSpec((H, BN, cp), lambda i: (0, i, 0)),
            pl.BlockSpec((H * cp, 48), lambda i: (0, 0)),
            pl.BlockSpec((H, cp, 48), lambda i: (0, 0, 0)),
            pl.BlockSpec((1, 48), lambda i: (0, 0)),
        ],
        out_specs=pl.BlockSpec((BN, 48), lambda i: (i, 0)),
        out_shape=jax.ShapeDtypeStruct((NP, 48), jnp.float32),
    )(h2n, nbrg, wgr, wgn, bg)


def _head_body(h3_ref, nbr_ref, deg_ref, wsn_ref, wsr_ref, bs_ref,
               batch_ref, fc1w_ref, fc1b_ref, fc3w_ref, fc3b_ref,
               out_ref, sums_ref, cnt_ref):
    i = pl.program_id(0)

    @pl.when(i == 0)
    def _():
        sums_ref[...] = jnp.zeros_like(sums_ref)
        cnt_ref[...] = jnp.zeros_like(cnt_ref)

    nm = nbr_ref[...] / jnp.maximum(deg_ref[...], 1.0)
    h4 = jax.nn.relu(
        jnp.dot(nm, wsn_ref[...], preferred_element_type=jnp.float32)
        + jnp.dot(h3_ref[...], wsr_ref[...],
                  preferred_element_type=jnp.float32)
        + bs_ref[...])
    onehot = (batch_ref[...] == lax.broadcasted_iota(jnp.int32, (1, G), 1)
              ).astype(jnp.float32)
    sums_ref[...] += lax.dot_general(onehot, h4, (((0,), (0,)), ((), ())),
                                     preferred_element_type=jnp.float32)
    cnt_ref[...] += jnp.sum(onehot, axis=0, keepdims=True)

    @pl.when(i == NB - 1)
    def _():
        g = sums_ref[...] / jnp.maximum(cnt_ref[...], 1.0).T
        g = jax.nn.relu(g @ fc1w_ref[...] + fc1b_ref[...])
        out_ref[...] = g @ fc3w_ref[...] + fc3b_ref[...]


def _head(h3, nbr3, deg, wsn, wsr, bs, batch, fc1w, fc1b, fc3w, fc3b):
    return pl.pallas_call(
        _head_body,
        grid=(NB,),
        in_specs=[
            pl.BlockSpec((BN, 48), lambda i: (i, 0)),
            pl.BlockSpec((BN, 48), lambda i: (i, 0)),
            pl.BlockSpec((BN, 1), lambda i: (i, 0)),
            pl.BlockSpec((48, 48), lambda i: (0, 0)),
            pl.BlockSpec((48, 48), lambda i: (0, 0)),
            pl.BlockSpec((1, 48), lambda i: (0, 0)),
            pl.BlockSpec((BN, 1), lambda i: (i, 0)),
            pl.BlockSpec((48, 32), lambda i: (0, 0)),
            pl.BlockSpec((1, 32), lambda i: (0, 0)),
            pl.BlockSpec((32, 10), lambda i: (0, 0)),
            pl.BlockSpec((1, 10), lambda i: (0, 0)),
        ],
        out_specs=pl.BlockSpec((G, 10), lambda i: (0, 0)),
        out_shape=jax.ShapeDtypeStruct((G, 10), jnp.float32),
        scratch_shapes=[
            pltpu.VMEM((G, 48), jnp.float32),
            pltpu.VMEM((1, G), jnp.float32),
        ],
    )(h3, nbr3, deg, wsn, wsr, bs, batch, fc1w, fc1b, fc3w, fc3b)


def _sc_softmax_body(src_hbm, dst_hbm, as_hbm, adm_hbm, z_hbm,
                     den_hbm, gidx_hbm, sidx_hbm,
                     srcb, dstb, asb, admb, exb, gib, sib, den_sh):
    c = lax.axis_index("c")
    s = lax.axis_index("s")
    rpt = NP // NTILE
    r0 = s * rpt
    pltpu.sync_copy(z_hbm.at[pl.ds(r0, rpt)], den_sh.at[pl.ds(r0, rpt)])
    plsc.subcore_barrier()
    epc = E // 2
    ept = epc // NTILE

    @pl.loop(0, ept // CH)
    def _(ci):
        base = c * epc + s * ept + ci * CH
        pltpu.sync_copy(src_hbm.at[pl.ds(base, CH)], srcb)
        pltpu.sync_copy(dst_hbm.at[pl.ds(base, CH)], dstb)
        pltpu.sync_copy(as_hbm.at[srcb], asb)
        pltpu.sync_copy(adm_hbm.at[dstb], admb)

        @plsc.parallel_loop(0, CH, unroll=4)
        def _(i):
            ei = asb[i, :] + admb[i, pl.ds(0, H)]
            e = jnp.maximum(ei, 0.2 * ei)
            exb[i, :] = jnp.exp(e - admb[i, pl.ds(H, H)])
            off = jnp.where(ei < 0.0, NP, 0).astype(jnp.int32)
            srcv = plsc.load_gather(srcb, [jnp.full((16,), i, jnp.int32)])
            dstv = plsc.load_gather(dstb, [jnp.full((16,), i, jnp.int32)])
            gib[i, :] = srcv + off
            sib[i, :] = dstv + off

        pltpu.sync_copy(exb, den_sh.at[dstb], add=True)
        pltpu.sync_copy(gib, gidx_hbm.at[pl.ds(base, CH)])
        pltpu.sync_copy(sib, sidx_hbm.at[pl.ds(base, CH)])

    plsc.subcore_barrier()

    @pl.when(c == 0)
    def _():
        pltpu.sync_copy(den_sh.at[pl.ds(r0, rpt)],
                        den_hbm.at[0, pl.ds(r0, rpt)])

    @pl.when(c == 1)
    def _():
        pltpu.sync_copy(den_sh.at[pl.ds(r0, rpt)],
                        den_hbm.at[1, pl.ds(r0, rpt)])


def _sc_softmax(src, dst, a_s, adm, z16):
    k = functools.partial(
        pl.kernel, mesh=_mesh, compiler_params=_sc_params,
        out_type=[jax.ShapeDtypeStruct((2, NP, H), jnp.float32),
                  jax.ShapeDtypeStruct((E, H), jnp.int32),
                  jax.ShapeDtypeStruct((E, H), jnp.int32)],
        scratch_types=[
            pltpu.VMEM((CH,), jnp.int32),
            pltpu.VMEM((CH,), jnp.int32),
            pltpu.VMEM((CH, H), jnp.float32),
            pltpu.VMEM((CH, 2 * H), jnp.float32),
            pltpu.VMEM((CH, H), jnp.float32),
            pltpu.VMEM((CH, H), jnp.int32),
            pltpu.VMEM((CH, H), jnp.int32),
            pltpu.VMEM_SHARED((NP, H), jnp.float32),
        ])(_sc_softmax_body)
    return k(src, dst, a_s, adm, z16)


def _make_stream_body(M, CHX, GRP, shared, cp):
    nch = (E // NTILE) // CHX
    ngroups = nch // GRP

    def body(t_hbm, g3_hbm, s3_hbm, z_hbm, out_hbm,
             gib, sib, rows, acc_sh, sg0, sg1):
        c = lax.axis_index("c")
        s = lax.axis_index("s")
        rpt = M // NTILE
        r0 = s * rpt
        q0 = s * nch

        sems = (sg0, sg1)

        @pl.loop(0, H // 2)
        def _(kl):
            k = 2 * kl + c
            kk = 0 if shared else k
            tref = t_hbm.at[k]

            def load_group(slot, gi):
                pltpu.sync_copy(g3_hbm.at[kk, pl.ds(q0 + gi * GRP, GRP)],
                                gib.at[slot])
                pltpu.sync_copy(s3_hbm.at[kk, pl.ds(q0 + gi * GRP, GRP)],
                                sib.at[slot])

            def start_gather(buf, slot, row):
                pltpu.async_copy(tref.at[gib.at[slot, row]],
                                 rows.at[buf], sems[buf])

            def wait_gather(buf, slot):
                pltpu.make_async_copy(tref.at[gib.at[slot, 0]],
                                      rows.at[buf], sems[buf]).wait()

            def scatter(buf, slot, row):
                pltpu.sync_copy(rows.at[buf], acc_sh.at[sib.at[slot, row]],
                                add=True)

            pltpu.sync_copy(z_hbm.at[pl.ds(r0, rpt)],
                            acc_sh.at[pl.ds(r0, rpt)])
            plsc.subcore_barrier()

            load_group(0, 0)
            start_gather(0, 0, 0)
            for g in range(ngroups):
                slot = g % 2

                @pl.loop(0, GRP // 2 - 1)
                def _(jj):
                    r = 2 * jj
                    start_gather(1, slot, r + 1)
                    wait_gather(0, slot)
                    scatter(0, slot, r)
                    start_gather(0, slot, r + 2)
                    wait_gather(1, slot)
                    scatter(1, slot, r + 1)

                r = GRP - 2
                start_gather(1, slot, r + 1)
                wait_gather(0, slot)
                scatter(0, slot, r)
                if g + 1 < ngroups:
                    load_group(1 - slot, g + 1)
                    start_gather(0, 1 - slot, 0)
                wait_gather(1, slot)
                scatter(1, slot, r + 1)

            plsc.subcore_barrier()
            pltpu.sync_copy(acc_sh.at[pl.ds(r0, rpt)],
                            out_hbm.at[k, pl.ds(r0, rpt)])

    return body


CHB2 = 200
CHC2 = 400
GRPB = 20
GRPC = 10


def _sc_gat_agg(t2, g3, s3, z2):
    cp = t2.shape[2]
    k = functools.partial(
        pl.kernel, mesh=_mesh, compiler_params=_sc_params,
        out_type=jax.ShapeDtypeStruct((H, 2 * NP, cp), jnp.float32),
        scratch_types=[
            pltpu.VMEM((2, GRPB, CHB2), jnp.int32),
            pltpu.VMEM((2, GRPB, CHB2), jnp.int32),
            pltpu.VMEM((2, CHB2, cp), jnp.float32),
            pltpu.VMEM_SHARED((2 * NP, cp), jnp.float32),
            pltpu.SemaphoreType.DMA,
            pltpu.SemaphoreType.DMA,
        ])(_make_stream_body(2 * NP, CHB2, GRPB, False, cp))
    return k(t2, g3, s3, z2)


def _sc_sum_agg(ht, src_e, dst_e, zc):
    cp = ht.shape[2]
    g3 = src_e.reshape(1, E // CHC2, CHC2)
    s3 = dst_e.reshape(1, E // CHC2, CHC2)
    k = functools.partial(
        pl.kernel, mesh=_mesh, compiler_params=_sc_params,
        out_type=jax.ShapeDtypeStruct((H, NP, cp), jnp.float32),
        scratch_types=[
            pltpu.VMEM((2, GRPC, CHC2), jnp.int32),
            pltpu.VMEM((2, GRPC, CHC2), jnp.int32),
            pltpu.VMEM((2, CHC2, cp), jnp.float32),
            pltpu.VMEM_SHARED((NP, cp), jnp.float32),
            pltpu.SemaphoreType.DMA,
            pltpu.SemaphoreType.DMA,
        ])(_make_stream_body(NP, CHC2, GRPC, True, cp))
    return k(ht, g3, s3, zc)


def _sc_sage_body(h3_hbm, src_hbm, dst_hbm, ones_hbm, z48_hbm, z16_hbm,
                  nbr_hbm, deg_hbm, srcb, dstb, rows, onesb,
                  nbr_sh, deg_sh):
    c = lax.axis_index("c")
    s = lax.axis_index("s")
    rpt = NP // NTILE
    r0 = s * rpt
    ept = E // NTILE

    @pl.when(c == 0)
    def _():
        pltpu.sync_copy(z48_hbm.at[pl.ds(r0, rpt)],
                        nbr_sh.at[pl.ds(r0, rpt)])
        plsc.subcore_barrier()

        @pl.loop(0, ept // CH)
        def _(ci):
            b = s * ept + ci * CH
            pltpu.sync_copy(src_hbm.at[pl.ds(b, CH)], srcb)
            pltpu.sync_copy(dst_hbm.at[pl.ds(b, CH)], dstb)
            pltpu.sync_copy(h3_hbm.at[srcb], rows)
            pltpu.sync_copy(rows, nbr_sh.at[dstb], add=True)

        plsc.subcore_barrier()
        pltpu.sync_copy(nbr_sh.at[pl.ds(r0, rpt)],
                        nbr_hbm.at[pl.ds(r0, rpt)])

    @pl.when(c == 1)
    def _():
        pltpu.sync_copy(z16_hbm.at[pl.ds(r0, rpt)],
                        deg_sh.at[pl.ds(r0, rpt)])
        pltpu.sync_copy(ones_hbm, onesb)
        plsc.subcore_barrier()

        @pl.loop(0, ept // CH)
        def _(ci):
            b = s * ept + ci * CH
            pltpu.sync_copy(dst_hbm.at[pl.ds(b, CH)], dstb)
            pltpu.sync_copy(onesb, deg_sh.at[dstb], add=True)

        plsc.subcore_barrier()
        pltpu.sync_copy(deg_sh.at[pl.ds(r0, rpt)],
                        deg_hbm.at[pl.ds(r0, rpt)])


def _sc_sage(h3, src, dst, ones, z48, z16):
    k = functools.partial(
        pl.kernel, mesh=_mesh, compiler_params=_sc_params,
        out_type=[jax.ShapeDtypeStruct((NP, 48), jnp.float32),
                  jax.ShapeDtypeStruct((NP, H), jnp.float32)],
        scratch_types=[
            pltpu.VMEM((CH,), jnp.int32),
            pltpu.VMEM((CH,), jnp.int32),
            pltpu.VMEM((CH, 48), jnp.float32),
            pltpu.VMEM((CH, H), jnp.float32),
            pltpu.VMEM_SHARED((N, 48), jnp.float32),
            pltpu.VMEM_SHARED((NP, H), jnp.float32),
        ])(_sc_sage_body)
    return k(h3, src, dst, ones, z48, z16)


def _gat_layer(x, src, dst, wt, asr, adr, bp, z16, z2):
    ht = _mm(x, wt)
    a_s, a_d = _asad(ht, asr, adr)
    adm, exself = _prep(a_s, a_d)
    den, gidx, sidx = _sc_softmax(src, dst, a_s, adm, z16)
    gidx_t = _tr(gidx)
    sidx_t = _tr(sidx)
    t2 = _scale_tables(ht, a_s).reshape(H, 2 * NP, ht.shape[2])
    acc = _sc_gat_agg(t2, gidx_t, sidx_t, z2).reshape(H, 2, NP, ht.shape[2])
    return _combine(acc, ht, exself, den, adm, bp)


def kernel(x, edge_index, batch, W1, att_src1, att_dst1, b1, W2, att_src2,
           att_dst2, b2, Wg_root, Wg_nbr, bg, Ws_nbr, Ws_root, bs, fc1_w,
           fc1_b, fc3_w, fc3_b):
    f32 = jnp.float32
    src = edge_index[0]
    dst = edge_index[1]
    C1 = att_src1.shape[1]
    C2 = att_src2.shape[1]

    def padh(a, c, cp):
        return jnp.pad(a.reshape(H, 1, c), ((0, 0), (0, 0), (0, cp - c)))

    w1t = jnp.pad(W1.reshape(-1, H, C1), ((0, 0), (0, 0), (0, CP - C1))
                  ).transpose(1, 0, 2)
    w2t = jnp.pad(W2.reshape(H, C1, H, C2),
                  ((0, 0), (0, CP - C1), (0, 0), (0, CP2 - C2))
                  ).reshape(H * CP, H, CP2).transpose(1, 0, 2)
    wgr = jnp.pad(Wg_root.reshape(H, C2, 40),
                  ((0, 0), (0, CP2 - C2), (0, 8))).reshape(H * CP2, 48)
    wgn = jnp.pad(Wg_nbr.reshape(H, C2, 40),
                  ((0, 0), (0, CP2 - C2), (0, 8)))
    wsn = jnp.pad(Ws_nbr, ((0, 8), (0, 8)))
    wsr = jnp.pad(Ws_root, ((0, 8), (0, 8)))
    fc1p = jnp.pad(fc1_w, ((0, 8), (0, 0)))

    z16 = jnp.zeros((NP, H), f32)
    z2a = jnp.zeros((2 * NP, CP), f32)
    z2b = jnp.zeros((2 * NP, CP2), f32)
    z48 = jnp.zeros((NP, 48), f32)
    ones = jnp.ones((CH, H), f32)

    x = jnp.pad(x, ((0, NP - N), (0, 0)))
    batch = jnp.pad(batch, (0, NP - N), constant_values=G)

    h2nm, _ = _gat_layer(x, src, dst, w1t, padh(att_src1, C1, CP),
                         padh(att_dst1, C1, CP), padh(b1, C1, CP), z16, z2a)
    h2 = h2nm.reshape(NP, H * CP)
    h3nm, h3hm = _gat_layer(h2, src, dst, w2t, padh(att_src2, C2, CP2),
                            padh(att_dst2, C2, CP2), padh(b2, C2, CP2),
                            z16, z2b)
    h3n = h3nm.reshape(NP, H * CP2)

    nbrg = _sc_sum_agg(h3hm, src, dst, z48)
    h4 = _graphconv(h3n, nbrg, wgr, wgn, jnp.pad(bg, (0, 8)).reshape(1, 48))

    nbr3, deg = _sc_sage(h4, src, dst, ones, z48, z16)
    out = _head(h4, nbr3, deg[:, :1], wsn, wsr,
                jnp.pad(bs, (0, 8)).reshape(1, 48), batch.reshape(NP, 1),
                fc1p, fc1_b.reshape(1, 32), fc3_w, fc3_b.reshape(1, 10))
    return out


# speedup vs baseline: 19.2982x; 1.0215x over previous
import functools

import jax
import jax.numpy as jnp
from jax import lax
from jax.experimental import pallas as pl
from jax.experimental.pallas import tpu as pltpu
from jax.experimental.pallas import tpu_sc as plsc

N = 10000
E = 320000
G = 64
H = 16
CP = 64
CP2 = 48
NP = 10240
BN = 512
NB = NP // BN
NTILE = 16
CH = 1000
CHB = 400
_mesh = plsc.VectorSubcoreMesh(core_axis_name="c", subcore_axis_name="s")
_sc_params = pltpu.CompilerParams(use_tc_tiling_on_sc=False, needs_layout_passes=False)


def _mm_body(x_ref, w_ref, ht_ref):
    ht_ref[0] = jnp.dot(x_ref[...], w_ref[0],
                        preferred_element_type=jnp.float32)


def _mm(x, wt):
    d = x.shape[1]
    cp = wt.shape[2]
    return pl.pallas_call(
        _mm_body,
        grid=(NB, H),
        in_specs=[
            pl.BlockSpec((BN, d), lambda i, k: (i, 0)),
            pl.BlockSpec((1, d, cp), lambda i, k: (k, 0, 0)),
        ],
        out_specs=pl.BlockSpec((1, BN, cp), lambda i, k: (k, i, 0)),
        out_shape=jax.ShapeDtypeStruct((H, NP, cp), jnp.float32),
    )(x, wt)


def _asad_body(ht_ref, asr_ref, adr_ref, as_ref, ad_ref):
    cols_s = []
    cols_d = []
    for k in range(H):
        h = ht_ref[k]
        cols_s.append(jnp.sum(h * asr_ref[k], axis=1, keepdims=True))
        cols_d.append(jnp.sum(h * adr_ref[k], axis=1, keepdims=True))
    as_ref[...] = jnp.concatenate(cols_s, axis=1)
    ad_ref[...] = jnp.concatenate(cols_d, axis=1)


def _asad(ht, asr, adr):
    cp = ht.shape[2]
    return pl.pallas_call(
        _asad_body,
        grid=(NB,),
        in_specs=[
            pl.BlockSpec((H, BN, cp), lambda i: (0, i, 0)),
            pl.BlockSpec((H, 1, cp), lambda i: (0, 0, 0)),
            pl.BlockSpec((H, 1, cp), lambda i: (0, 0, 0)),
        ],
        out_specs=[
            pl.BlockSpec((BN, H), lambda i: (i, 0)),
            pl.BlockSpec((BN, H), lambda i: (i, 0)),
        ],
        out_shape=[
            jax.ShapeDtypeStruct((NP, H), jnp.float32),
            jax.ShapeDtypeStruct((NP, H), jnp.float32),
        ],
    )(ht, asr, adr)


def _prep_body(as_ref, ad_ref, adm_ref, exself_ref):
    a_s = as_ref[...]
    a_d = ad_ref[...]
    mb = jnp.max(a_s, axis=0, keepdims=True) + a_d
    m = jnp.maximum(mb, 0.2 * mb)
    adm_ref[:, :H] = a_d
    adm_ref[:, H:] = m
    e = a_s + a_d
    e = jnp.maximum(e, 0.2 * e)
    exself_ref[...] = jnp.exp(e - m)


def _prep(a_s, a_d):
    return pl.pallas_call(
        _prep_body,
        out_shape=[
            jax.ShapeDtypeStruct((NP, 2 * H), jnp.float32),
            jax.ShapeDtypeStruct((NP, H), jnp.float32),
        ],
    )(a_s, a_d)


def _tr_body(x_ref, o_ref):
    o_ref[...] = x_ref[...].T


def _tr(ex):
    be = 6400
    return pl.pallas_call(
        _tr_body,
        grid=(E // be,),
        in_specs=[pl.BlockSpec((be, H), lambda i: (i, 0))],
        out_specs=pl.BlockSpec((H, be), lambda i: (0, i)),
        out_shape=jax.ShapeDtypeStruct((H, E), ex.dtype),
    )(ex).reshape(H, E // CHB2, CHB2)


def _scale_body(ht_ref, as_ref, t2_ref):
    a_s = as_ref[...]
    for k in range(H):
        h = ht_ref[k]
        t2_ref[k, 0] = h * jnp.exp(a_s[:, k:k + 1])
        t2_ref[k, 1] = h * jnp.exp(0.2 * a_s[:, k:k + 1])


def _scale_tables(ht, a_s):
    cp = ht.shape[2]
    return pl.pallas_call(
        _scale_body,
        grid=(NB,),
        in_specs=[
            pl.BlockSpec((H, BN, cp), lambda i: (0, i, 0)),
            pl.BlockSpec((BN, H), lambda i: (i, 0)),
        ],
        out_specs=pl.BlockSpec((H, 2, BN, cp), lambda i: (0, 0, i, 0)),
        out_shape=jax.ShapeDtypeStruct((H, 2, NP, cp), jnp.float32),
    )(ht, a_s)


def _combine_body(accp_ref, accn_ref, ht_ref, exself_ref, den_ref,
                  adm_ref, b_ref, nm_ref, hm_ref):
    exs = exself_ref[...]
    adm = adm_ref[...]
    den = den_ref[0] + den_ref[1] + exs + 1e-16
    for k in range(H):
        ad_k = adm[:, k:k + 1]
        m_k = adm[:, H + k:H + k + 1]
        num = (jnp.exp(ad_k - m_k) * accp_ref[k, 0]
               + jnp.exp(0.2 * ad_k - m_k) * accn_ref[k, 0]
               + exs[:, k:k + 1] * ht_ref[k])
        r = jax.nn.relu(num / den[:, k:k + 1] + b_ref[k])
        nm_ref[:, k, :] = r
        hm_ref[k] = r


def _combine(acc, ht, exself, den, adm, b):
    cp = ht.shape[2]
    return pl.pallas_call(
        _combine_body,
        grid=(NB,),
        in_specs=[
            pl.BlockSpec((H, 1, BN, cp), lambda i: (0, 0, i, 0)),
            pl.BlockSpec((H, 1, BN, cp), lambda i: (0, 1, i, 0)),
            pl.BlockSpec((H, BN, cp), lambda i: (0, i, 0)),
            pl.BlockSpec((BN, H), lambda i: (i, 0)),
            pl.BlockSpec((2, BN, H), lambda i: (0, i, 0)),
            pl.BlockSpec((BN, 2 * H), lambda i: (i, 0)),
            pl.BlockSpec((H, 1, cp), lambda i: (0, 0, 0)),
        ],
        out_specs=[
            pl.BlockSpec((BN, H, cp), lambda i: (i, 0, 0)),
            pl.BlockSpec((H, BN, cp), lambda i: (0, i, 0)),
        ],
        out_shape=[
            jax.ShapeDtypeStruct((NP, H, cp), jnp.float32),
            jax.ShapeDtypeStruct((H, NP, cp), jnp.float32),
        ],
    )(acc, acc, ht, exself, den, adm, b)


def _graphconv_body(h2_ref, nbr_ref, wgr_ref, wgn_ref, bg_ref, o_ref):
    acc = jnp.dot(h2_ref[...], wgr_ref[...],
                  preferred_element_type=jnp.float32) + bg_ref[...]
    for k in range(H):
        acc += jnp.dot(nbr_ref[k], wgn_ref[k],
                       preferred_element_type=jnp.float32)
    o_ref[...] = jax.nn.relu(acc)


def _graphconv(h2n, nbrg, wgr, wgn, bg):
    cp = nbrg.shape[2]
    return pl.pallas_call(
        _graphconv_body,
        grid=(NB,),
        in_specs=[
            pl.BlockSpec((BN, H * cp), lambda i: (i, 0)),
            pl.BlockSpec((H, BN, cp), lambda i: (0, i, 0)),
            pl.BlockSpec((H * cp, 48), lambda i: (0, 0)),
            pl.BlockSpec((H, cp, 48), lambda i: (0, 0, 0)),
            pl.BlockSpec((1, 48), lambda i: (0, 0)),
        ],
        out_specs=pl.BlockSpec((BN, 48), lambda i: (i, 0)),
        out_shape=jax.ShapeDtypeStruct((NP, 48), jnp.float32),
    )(h2n, nbrg, wgr, wgn, bg)


def _head_body(h3_ref, nbr_ref, deg_ref, wsn_ref, wsr_ref, bs_ref,
               batch_ref, fc1w_ref, fc1b_ref, fc3w_ref, fc3b_ref,
               out_ref, sums_ref, cnt_ref):
    i = pl.program_id(0)

    @pl.when(i == 0)
    def _():
        sums_ref[...] = jnp.zeros_like(sums_ref)
        cnt_ref[...] = jnp.zeros_like(cnt_ref)

    nm = nbr_ref[...] / jnp.maximum(deg_ref[...], 1.0)
    h4 = jax.nn.relu(
        jnp.dot(nm, wsn_ref[...], preferred_element_type=jnp.float32)
        + jnp.dot(h3_ref[...], wsr_ref[...],
                  preferred_element_type=jnp.float32)
        + bs_ref[...])
    onehot = (batch_ref[...] == lax.broadcasted_iota(jnp.int32, (1, G), 1)
              ).astype(jnp.float32)
    sums_ref[...] += lax.dot_general(onehot, h4, (((0,), (0,)), ((), ())),
                                     preferred_element_type=jnp.float32)
    cnt_ref[...] += jnp.sum(onehot, axis=0, keepdims=True)

    @pl.when(i == NB - 1)
    def _():
        g = sums_ref[...] / jnp.maximum(cnt_ref[...], 1.0).T
        g = jax.nn.relu(g @ fc1w_ref[...] + fc1b_ref[...])
        out_ref[...] = g @ fc3w_ref[...] + fc3b_ref[...]


def _head(h3, nbr3, deg, wsn, wsr, bs, batch, fc1w, fc1b, fc3w, fc3b):
    return pl.pallas_call(
        _head_body,
        grid=(NB,),
        in_specs=[
            pl.BlockSpec((BN, 48), lambda i: (i, 0)),
            pl.BlockSpec((BN, 48), lambda i: (i, 0)),
            pl.BlockSpec((BN, 1), lambda i: (i, 0)),
            pl.BlockSpec((48, 48), lambda i: (0, 0)),
            pl.BlockSpec((48, 48), lambda i: (0, 0)),
            pl.BlockSpec((1, 48), lambda i: (0, 0)),
            pl.BlockSpec((BN, 1), lambda i: (i, 0)),
            pl.BlockSpec((48, 32), lambda i: (0, 0)),
            pl.BlockSpec((1, 32), lambda i: (0, 0)),
            pl.BlockSpec((32, 10), lambda i: (0, 0)),
            pl.BlockSpec((1, 10), lambda i: (0, 0)),
        ],
        out_specs=pl.BlockSpec((G, 10), lambda i: (0, 0)),
        out_shape=jax.ShapeDtypeStruct((G, 10), jnp.float32),
        scratch_shapes=[
            pltpu.VMEM((G, 48), jnp.float32),
            pltpu.VMEM((1, G), jnp.float32),
        ],
    )(h3, nbr3, deg, wsn, wsr, bs, batch, fc1w, fc1b, fc3w, fc3b)


def _sc_softmax_body(src_hbm, dst_hbm, as_hbm, adm_hbm, z_hbm,
                     den_hbm, gidx_hbm, sidx_hbm,
                     srcb, dstb, asb, admb, exb, gib, sib, den_sh):
    c = lax.axis_index("c")
    s = lax.axis_index("s")
    rpt = NP // NTILE
    r0 = s * rpt
    pltpu.sync_copy(z_hbm.at[pl.ds(r0, rpt)], den_sh.at[pl.ds(r0, rpt)])
    plsc.subcore_barrier()
    epc = E // 2
    ept = epc // NTILE

    @pl.loop(0, ept // CH)
    def _(ci):
        base = c * epc + s * ept + ci * CH
        pltpu.sync_copy(src_hbm.at[pl.ds(base, CH)], srcb)
        pltpu.sync_copy(dst_hbm.at[pl.ds(base, CH)], dstb)
        pltpu.sync_copy(as_hbm.at[srcb], asb)
        pltpu.sync_copy(adm_hbm.at[dstb], admb)

        @plsc.parallel_loop(0, CH, unroll=4)
        def _(i):
            ei = asb[i, :] + admb[i, pl.ds(0, H)]
            e = jnp.maximum(ei, 0.2 * ei)
            exb[i, :] = jnp.exp(e - admb[i, pl.ds(H, H)])
            off = jnp.where(ei < 0.0, NP, 0).astype(jnp.int32)
            srcv = plsc.load_gather(srcb, [jnp.full((16,), i, jnp.int32)])
            dstv = plsc.load_gather(dstb, [jnp.full((16,), i, jnp.int32)])
            gib[i, :] = srcv + off
            sib[i, :] = dstv + off

        pltpu.sync_copy(exb, den_sh.at[dstb], add=True)
        pltpu.sync_copy(gib, gidx_hbm.at[pl.ds(base, CH)])
        pltpu.sync_copy(sib, sidx_hbm.at[pl.ds(base, CH)])

    plsc.subcore_barrier()

    @pl.when(c == 0)
    def _():
        pltpu.sync_copy(den_sh.at[pl.ds(r0, rpt)],
                        den_hbm.at[0, pl.ds(r0, rpt)])

    @pl.when(c == 1)
    def _():
        pltpu.sync_copy(den_sh.at[pl.ds(r0, rpt)],
                        den_hbm.at[1, pl.ds(r0, rpt)])


def _sc_softmax(src, dst, a_s, adm, z16):
    k = functools.partial(
        pl.kernel, mesh=_mesh, compiler_params=_sc_params,
        out_type=[jax.ShapeDtypeStruct((2, NP, H), jnp.float32),
                  jax.ShapeDtypeStruct((E, H), jnp.int32),
                  jax.ShapeDtypeStruct((E, H), jnp.int32)],
        scratch_types=[
            pltpu.VMEM((CH,), jnp.int32),
            pltpu.VMEM((CH,), jnp.int32),
            pltpu.VMEM((CH, H), jnp.float32),
            pltpu.VMEM((CH, 2 * H), jnp.float32),
            pltpu.VMEM((CH, H), jnp.float32),
            pltpu.VMEM((CH, H), jnp.int32),
            pltpu.VMEM((CH, H), jnp.int32),
            pltpu.VMEM_SHARED((NP, H), jnp.float32),
        ])(_sc_softmax_body)
    return k(src, dst, a_s, adm, z16)


def _make_stream_body(M, CHX, GRP, shared, cp):
    nch = (E // NTILE) // CHX
    ngroups = nch // GRP

    def body(t_hbm, g3_hbm, s3_hbm, z_hbm, out_hbm,
             gib, sib, rows, acc_sh, sg0, sg1, si0, si1):
        c = lax.axis_index("c")
        s = lax.axis_index("s")
        rpt = M // NTILE
        r0 = s * rpt
        q0 = s * nch

        sems = (sg0, sg1)

        @pl.loop(0, H // 2)
        def _(kl):
            k = 2 * kl + c
            kk = 0 if shared else k
            tref = t_hbm.at[k]

            isems = (si0, si1)

            def start_load_group(slot, gi):
                pltpu.async_copy(g3_hbm.at[kk, pl.ds(q0 + gi * GRP, GRP)],
                                 gib.at[slot], isems[slot])
                pltpu.async_copy(s3_hbm.at[kk, pl.ds(q0 + gi * GRP, GRP)],
                                 sib.at[slot], isems[slot])

            def wait_load_group(slot):
                pltpu.make_async_copy(g3_hbm.at[kk, pl.ds(q0, GRP)],
                                      gib.at[slot], isems[slot]).wait()
                pltpu.make_async_copy(s3_hbm.at[kk, pl.ds(q0, GRP)],
                                      sib.at[slot], isems[slot]).wait()

            def start_gather(buf, slot, row):
                pltpu.async_copy(tref.at[gib.at[slot, row]],
                                 rows.at[buf], sems[buf])

            def wait_gather(buf, slot):
                pltpu.make_async_copy(tref.at[gib.at[slot, 0]],
                                      rows.at[buf], sems[buf]).wait()

            def scatter(buf, slot, row):
                pltpu.sync_copy(rows.at[buf], acc_sh.at[sib.at[slot, row]],
                                add=True)

            pltpu.sync_copy(z_hbm.at[pl.ds(r0, rpt)],
                            acc_sh.at[pl.ds(r0, rpt)])
            plsc.subcore_barrier()

            start_load_group(0, 0)
            wait_load_group(0)
            start_gather(0, 0, 0)
            for g in range(ngroups):
                slot = g % 2
                if g + 1 < ngroups:
                    start_load_group(1 - slot, g + 1)

                @pl.loop(0, GRP // 2 - 1)
                def _(jj):
                    r = 2 * jj
                    start_gather(1, slot, r + 1)
                    wait_gather(0, slot)
                    scatter(0, slot, r)
                    start_gather(0, slot, r + 2)
                    wait_gather(1, slot)
                    scatter(1, slot, r + 1)

                r = GRP - 2
                start_gather(1, slot, r + 1)
                wait_gather(0, slot)
                scatter(0, slot, r)
                if g + 1 < ngroups:
                    wait_load_group(1 - slot)
                    start_gather(0, 1 - slot, 0)
                wait_gather(1, slot)
                scatter(1, slot, r + 1)

            plsc.subcore_barrier()
            pltpu.sync_copy(acc_sh.at[pl.ds(r0, rpt)],
                            out_hbm.at[k, pl.ds(r0, rpt)])

    return body


CHB2 = 200
CHC2 = 400
GRPB = 20
GRPC = 10


def _sc_gat_agg(t2, g3, s3, z2):
    cp = t2.shape[2]
    k = functools.partial(
        pl.kernel, mesh=_mesh, compiler_params=_sc_params,
        out_type=jax.ShapeDtypeStruct((H, 2 * NP, cp), jnp.float32),
        scratch_types=[
            pltpu.VMEM((2, GRPB, CHB2), jnp.int32),
            pltpu.VMEM((2, GRPB, CHB2), jnp.int32),
            pltpu.VMEM((2, CHB2, cp), jnp.float32),
            pltpu.VMEM_SHARED((2 * NP, cp), jnp.float32),
            pltpu.SemaphoreType.DMA,
            pltpu.SemaphoreType.DMA,
            pltpu.SemaphoreType.DMA,
            pltpu.SemaphoreType.DMA,
        ])(_make_stream_body(2 * NP, CHB2, GRPB, False, cp))
    return k(t2, g3, s3, z2)


def _sc_sum_agg(ht, src_e, dst_e, zc):
    cp = ht.shape[2]
    g3 = src_e.reshape(1, E // CHC2, CHC2)
    s3 = dst_e.reshape(1, E // CHC2, CHC2)
    k = functools.partial(
        pl.kernel, mesh=_mesh, compiler_params=_sc_params,
        out_type=jax.ShapeDtypeStruct((H, NP, cp), jnp.float32),
        scratch_types=[
            pltpu.VMEM((2, GRPC, CHC2), jnp.int32),
            pltpu.VMEM((2, GRPC, CHC2), jnp.int32),
            pltpu.VMEM((2, CHC2, cp), jnp.float32),
            pltpu.VMEM_SHARED((NP, cp), jnp.float32),
            pltpu.SemaphoreType.DMA,
            pltpu.SemaphoreType.DMA,
            pltpu.SemaphoreType.DMA,
            pltpu.SemaphoreType.DMA,
        ])(_make_stream_body(NP, CHC2, GRPC, True, cp))
    return k(ht, g3, s3, zc)


def _sc_sage_body(h3_hbm, src_hbm, dst_hbm, ones_hbm, z48_hbm, z16_hbm,
                  nbr_hbm, deg_hbm, srcb, dstb, rows, onesb,
                  nbr_sh, deg_sh):
    c = lax.axis_index("c")
    s = lax.axis_index("s")
    rpt = NP // NTILE
    r0 = s * rpt
    ept = E // NTILE

    @pl.when(c == 0)
    def _():
        pltpu.sync_copy(z48_hbm.at[pl.ds(r0, rpt)],
                        nbr_sh.at[pl.ds(r0, rpt)])
        plsc.subcore_barrier()

        @pl.loop(0, ept // CH)
        def _(ci):
            b = s * ept + ci * CH
            pltpu.sync_copy(src_hbm.at[pl.ds(b, CH)], srcb)
            pltpu.sync_copy(dst_hbm.at[pl.ds(b, CH)], dstb)
            pltpu.sync_copy(h3_hbm.at[srcb], rows)
            pltpu.sync_copy(rows, nbr_sh.at[dstb], add=True)

        plsc.subcore_barrier()
        pltpu.sync_copy(nbr_sh.at[pl.ds(r0, rpt)],
                        nbr_hbm.at[pl.ds(r0, rpt)])

    @pl.when(c == 1)
    def _():
        pltpu.sync_copy(z16_hbm.at[pl.ds(r0, rpt)],
                        deg_sh.at[pl.ds(r0, rpt)])
        pltpu.sync_copy(ones_hbm, onesb)
        plsc.subcore_barrier()

        @pl.loop(0, ept // CH)
        def _(ci):
            b = s * ept + ci * CH
            pltpu.sync_copy(dst_hbm.at[pl.ds(b, CH)], dstb)
            pltpu.sync_copy(onesb, deg_sh.at[dstb], add=True)

        plsc.subcore_barrier()
        pltpu.sync_copy(deg_sh.at[pl.ds(r0, rpt)],
                        deg_hbm.at[pl.ds(r0, rpt)])


def _sc_sage(h3, src, dst, ones, z48, z16):
    k = functools.partial(
        pl.kernel, mesh=_mesh, compiler_params=_sc_params,
        out_type=[jax.ShapeDtypeStruct((NP, 48), jnp.float32),
                  jax.ShapeDtypeStruct((NP, H), jnp.float32)],
        scratch_types=[
            pltpu.VMEM((CH,), jnp.int32),
            pltpu.VMEM((CH,), jnp.int32),
            pltpu.VMEM((CH, 48), jnp.float32),
            pltpu.VMEM((CH, H), jnp.float32),
            pltpu.VMEM_SHARED((N, 48), jnp.float32),
            pltpu.VMEM_SHARED((NP, H), jnp.float32),
        ])(_sc_sage_body)
    return k(h3, src, dst, ones, z48, z16)


def _gat_layer(x, src, dst, wt, asr, adr, bp, z16, z2):
    ht = _mm(x, wt)
    a_s, a_d = _asad(ht, asr, adr)
    adm, exself = _prep(a_s, a_d)
    den, gidx, sidx = _sc_softmax(src, dst, a_s, adm, z16)
    gidx_t = _tr(gidx)
    sidx_t = _tr(sidx)
    t2 = _scale_tables(ht, a_s).reshape(H, 2 * NP, ht.shape[2])
    acc = _sc_gat_agg(t2, gidx_t, sidx_t, z2).reshape(H, 2, NP, ht.shape[2])
    return _combine(acc, ht, exself, den, adm, bp)


def kernel(x, edge_index, batch, W1, att_src1, att_dst1, b1, W2, att_src2,
           att_dst2, b2, Wg_root, Wg_nbr, bg, Ws_nbr, Ws_root, bs, fc1_w,
           fc1_b, fc3_w, fc3_b):
    f32 = jnp.float32
    src = edge_index[0]
    dst = edge_index[1]
    C1 = att_src1.shape[1]
    C2 = att_src2.shape[1]

    def padh(a, c, cp):
        return jnp.pad(a.reshape(H, 1, c), ((0, 0), (0, 0), (0, cp - c)))

    w1t = jnp.pad(W1.reshape(-1, H, C1), ((0, 0), (0, 0), (0, CP - C1))
                  ).transpose(1, 0, 2)
    w2t = jnp.pad(W2.reshape(H, C1, H, C2),
                  ((0, 0), (0, CP - C1), (0, 0), (0, CP2 - C2))
                  ).reshape(H * CP, H, CP2).transpose(1, 0, 2)
    wgr = jnp.pad(Wg_root.reshape(H, C2, 40),
                  ((0, 0), (0, CP2 - C2), (0, 8))).reshape(H * CP2, 48)
    wgn = jnp.pad(Wg_nbr.reshape(H, C2, 40),
                  ((0, 0), (0, CP2 - C2), (0, 8)))
    wsn = jnp.pad(Ws_nbr, ((0, 8), (0, 8)))
    wsr = jnp.pad(Ws_root, ((0, 8), (0, 8)))
    fc1p = jnp.pad(fc1_w, ((0, 8), (0, 0)))

    z16 = jnp.zeros((NP, H), f32)
    z2a = jnp.zeros((2 * NP, CP), f32)
    z2b = jnp.zeros((2 * NP, CP2), f32)
    z48 = jnp.zeros((NP, 48), f32)
    ones = jnp.ones((CH, H), f32)

    x = jnp.pad(x, ((0, NP - N), (0, 0)))
    batch = jnp.pad(batch, (0, NP - N), constant_values=G)

    h2nm, _ = _gat_layer(x, src, dst, w1t, padh(att_src1, C1, CP),
                         padh(att_dst1, C1, CP), padh(b1, C1, CP), z16, z2a)
    h2 = h2nm.reshape(NP, H * CP)
    h3nm, h3hm = _gat_layer(h2, src, dst, w2t, padh(att_src2, C2, CP2),
                            padh(att_dst2, C2, CP2), padh(b2, C2, CP2),
                            z16, z2b)
    h3n = h3nm.reshape(NP, H * CP2)

    nbrg = _sc_sum_agg(h3hm, src, dst, z48)
    h4 = _graphconv(h3n, nbrg, wgr, wgn, jnp.pad(bg, (0, 8)).reshape(1, 48))

    nbr3, deg = _sc_sage(h4, src, dst, ones, z48, z16)
    out = _head(h4, nbr3, deg[:, :1], wsn, wsr,
                jnp.pad(bs, (0, 8)).reshape(1, 48), batch.reshape(NP, 1),
                fc1p, fc1_b.reshape(1, 32), fc3_w, fc3_b.reshape(1, 10))
    return out


# speedup vs baseline: 19.6747x; 1.0195x over previous
import functools

import jax
import jax.numpy as jnp
from jax import lax
from jax.experimental import pallas as pl
from jax.experimental.pallas import tpu as pltpu
from jax.experimental.pallas import tpu_sc as plsc

N = 10000
E = 320000
G = 64
H = 16
CP = 64
CP2 = 48
NP = 10240
BN = 512
NB = NP // BN
NTILE = 16
CH = 1000
CHB = 400
_mesh = plsc.VectorSubcoreMesh(core_axis_name="c", subcore_axis_name="s")
_sc_params = pltpu.CompilerParams(use_tc_tiling_on_sc=False, needs_layout_passes=False)


def _mm_body(x_ref, w_ref, ht_ref):
    ht_ref[0] = jnp.dot(x_ref[...], w_ref[0],
                        preferred_element_type=jnp.float32)


def _mm(x, wt):
    d = x.shape[1]
    cp = wt.shape[2]
    return pl.pallas_call(
        _mm_body,
        grid=(NB, H),
        in_specs=[
            pl.BlockSpec((BN, d), lambda i, k: (i, 0)),
            pl.BlockSpec((1, d, cp), lambda i, k: (k, 0, 0)),
        ],
        out_specs=pl.BlockSpec((1, BN, cp), lambda i, k: (k, i, 0)),
        out_shape=jax.ShapeDtypeStruct((H, NP, cp), jnp.float32),
    )(x, wt)


def _asad_body(ht_ref, asr_ref, adr_ref, as_ref, ad_ref):
    cols_s = []
    cols_d = []
    for k in range(H):
        h = ht_ref[k]
        cols_s.append(jnp.sum(h * asr_ref[k], axis=1, keepdims=True))
        cols_d.append(jnp.sum(h * adr_ref[k], axis=1, keepdims=True))
    as_ref[...] = jnp.concatenate(cols_s, axis=1)
    ad_ref[...] = jnp.concatenate(cols_d, axis=1)


def _asad(ht, asr, adr):
    cp = ht.shape[2]
    return pl.pallas_call(
        _asad_body,
        grid=(NB,),
        in_specs=[
            pl.BlockSpec((H, BN, cp), lambda i: (0, i, 0)),
            pl.BlockSpec((H, 1, cp), lambda i: (0, 0, 0)),
            pl.BlockSpec((H, 1, cp), lambda i: (0, 0, 0)),
        ],
        out_specs=[
            pl.BlockSpec((BN, H), lambda i: (i, 0)),
            pl.BlockSpec((BN, H), lambda i: (i, 0)),
        ],
        out_shape=[
            jax.ShapeDtypeStruct((NP, H), jnp.float32),
            jax.ShapeDtypeStruct((NP, H), jnp.float32),
        ],
    )(ht, asr, adr)


def _prep_body(as_ref, ad_ref, adm_ref, exself_ref):
    a_s = as_ref[...]
    a_d = ad_ref[...]
    mb = jnp.max(a_s, axis=0, keepdims=True) + a_d
    m = jnp.maximum(mb, 0.2 * mb)
    adm_ref[:, :H] = a_d
    adm_ref[:, H:] = m
    e = a_s + a_d
    e = jnp.maximum(e, 0.2 * e)
    exself_ref[...] = jnp.exp(e - m)


def _prep(a_s, a_d):
    return pl.pallas_call(
        _prep_body,
        out_shape=[
            jax.ShapeDtypeStruct((NP, 2 * H), jnp.float32),
            jax.ShapeDtypeStruct((NP, H), jnp.float32),
        ],
    )(a_s, a_d)


def _tr_body(x_ref, o_ref):
    o_ref[...] = x_ref[...].T


def _tr(ex):
    be = 6400
    return pl.pallas_call(
        _tr_body,
        grid=(E // be,),
        in_specs=[pl.BlockSpec((be, H), lambda i: (i, 0))],
        out_specs=pl.BlockSpec((H, be), lambda i: (0, i)),
        out_shape=jax.ShapeDtypeStruct((H, E), ex.dtype),
    )(ex).reshape(H, E // CHB2, CHB2)


def _scale_body(ht_ref, as_ref, t2_ref):
    fac = jnp.where(pl.program_id(0) < NB, 1.0, 0.2).astype(jnp.float32)
    a_s = as_ref[...]
    for k in range(H):
        t2_ref[k] = ht_ref[k] * jnp.exp(fac * a_s[:, k:k + 1])


def _scale_tables(ht, a_s):
    cp = ht.shape[2]
    return pl.pallas_call(
        _scale_body,
        grid=(2 * NB,),
        in_specs=[
            pl.BlockSpec((H, BN, cp), lambda j: (0, j % NB, 0)),
            pl.BlockSpec((BN, H), lambda j: (j % NB, 0)),
        ],
        out_specs=pl.BlockSpec((H, BN, cp), lambda j: (0, j, 0)),
        out_shape=jax.ShapeDtypeStruct((H, 2 * NP, cp), jnp.float32),
    )(ht, a_s)


def _combine_body(accp_ref, accn_ref, ht_ref, exself_ref, den_ref,
                  adm_ref, b_ref, nm_ref, hm_ref):
    exs = exself_ref[...]
    adm = adm_ref[...]
    den = den_ref[0] + den_ref[1] + exs + 1e-16
    cols = []
    for k in range(H):
        ad_k = adm[:, k:k + 1]
        m_k = adm[:, H + k:H + k + 1]
        num = (jnp.exp(ad_k - m_k) * accp_ref[k, 0]
               + jnp.exp(0.2 * ad_k - m_k) * accn_ref[k, 0]
               + exs[:, k:k + 1] * ht_ref[k])
        r = jax.nn.relu(num / den[:, k:k + 1] + b_ref[k])
        cols.append(r)
        hm_ref[k] = r
    nm_ref[...] = jnp.concatenate(cols, axis=1)


def _combine(acc, ht, exself, den, adm, b):
    cp = ht.shape[2]
    return pl.pallas_call(
        _combine_body,
        grid=(NB,),
        in_specs=[
            pl.BlockSpec((H, 1, BN, cp), lambda i: (0, 0, i, 0)),
            pl.BlockSpec((H, 1, BN, cp), lambda i: (0, 1, i, 0)),
            pl.BlockSpec((H, BN, cp), lambda i: (0, i, 0)),
            pl.BlockSpec((BN, H), lambda i: (i, 0)),
            pl.BlockSpec((2, BN, H), lambda i: (0, i, 0)),
            pl.BlockSpec((BN, 2 * H), lambda i: (i, 0)),
            pl.BlockSpec((H, 1, cp), lambda i: (0, 0, 0)),
        ],
        out_specs=[
            pl.BlockSpec((BN, H * cp), lambda i: (i, 0)),
            pl.BlockSpec((H, BN, cp), lambda i: (0, i, 0)),
        ],
        out_shape=[
            jax.ShapeDtypeStruct((NP, H * cp), jnp.float32),
            jax.ShapeDtypeStruct((H, NP, cp), jnp.float32),
        ],
    )(acc, acc, ht, exself, den, adm, b)


def _graphconv_body(h2_ref, nbr_ref, wgr_ref, wgn_ref, bg_ref, o_ref):
    acc = jnp.dot(h2_ref[...], wgr_ref[...],
                  preferred_element_type=jnp.float32) + bg_ref[...]
    for k in range(H):
        acc += jnp.dot(nbr_ref[k], wgn_ref[k],
                       preferred_element_type=jnp.float32)
    o_ref[...] = jax.nn.relu(acc)


def _graphconv(h2n, nbrg, wgr, wgn, bg):
    cp = nbrg.shape[2]
    return pl.pallas_call(
        _graphconv_body,
        grid=(NB,),
        in_specs=[
            pl.BlockSpec((BN, H * cp), lambda i: (i, 0)),
            pl.BlockSpec((H, BN, cp), lambda i: (0, i, 0)),
            pl.BlockSpec((H * cp, 48), lambda i: (0, 0)),
            pl.BlockSpec((H, cp, 48), lambda i: (0, 0, 0)),
            pl.BlockSpec((1, 48), lambda i: (0, 0)),
        ],
        out_specs=pl.BlockSpec((BN, 48), lambda i: (i, 0)),
        out_shape=jax.ShapeDtypeStruct((NP, 48), jnp.float32),
    )(h2n, nbrg, wgr, wgn, bg)


def _head_body(h3_ref, nbr_ref, deg_ref, wsn_ref, wsr_ref, bs_ref,
               batch_ref, fc1w_ref, fc1b_ref, fc3w_ref, fc3b_ref,
               out_ref, sums_ref, cnt_ref):
    i = pl.program_id(0)

    @pl.when(i == 0)
    def _():
        sums_ref[...] = jnp.zeros_like(sums_ref)
        cnt_ref[...] = jnp.zeros_like(cnt_ref)

    nm = nbr_ref[...] / jnp.maximum(deg_ref[...], 1.0)
    h4 = jax.nn.relu(
        jnp.dot(nm, wsn_ref[...], preferred_element_type=jnp.float32)
        + jnp.dot(h3_ref[...], wsr_ref[...],
                  preferred_element_type=jnp.float32)
        + bs_ref[...])
    onehot = (batch_ref[...] == lax.broadcasted_iota(jnp.int32, (1, G), 1)
              ).astype(jnp.float32)
    sums_ref[...] += lax.dot_general(onehot, h4, (((0,), (0,)), ((), ())),
                                     preferred_element_type=jnp.float32)
    cnt_ref[...] += jnp.sum(onehot, axis=0, keepdims=True)

    @pl.when(i == NB - 1)
    def _():
        g = sums_ref[...] / jnp.maximum(cnt_ref[...], 1.0).T
        g = jax.nn.relu(g @ fc1w_ref[...] + fc1b_ref[...])
        out_ref[...] = g @ fc3w_ref[...] + fc3b_ref[...]


def _head(h3, nbr3, deg, wsn, wsr, bs, batch, fc1w, fc1b, fc3w, fc3b):
    return pl.pallas_call(
        _head_body,
        grid=(NB,),
        in_specs=[
            pl.BlockSpec((BN, 48), lambda i: (i, 0)),
            pl.BlockSpec((BN, 48), lambda i: (i, 0)),
            pl.BlockSpec((BN, 1), lambda i: (i, 0)),
            pl.BlockSpec((48, 48), lambda i: (0, 0)),
            pl.BlockSpec((48, 48), lambda i: (0, 0)),
            pl.BlockSpec((1, 48), lambda i: (0, 0)),
            pl.BlockSpec((BN, 1), lambda i: (i, 0)),
            pl.BlockSpec((48, 32), lambda i: (0, 0)),
            pl.BlockSpec((1, 32), lambda i: (0, 0)),
            pl.BlockSpec((32, 10), lambda i: (0, 0)),
            pl.BlockSpec((1, 10), lambda i: (0, 0)),
        ],
        out_specs=pl.BlockSpec((G, 10), lambda i: (0, 0)),
        out_shape=jax.ShapeDtypeStruct((G, 10), jnp.float32),
        scratch_shapes=[
            pltpu.VMEM((G, 48), jnp.float32),
            pltpu.VMEM((1, G), jnp.float32),
        ],
    )(h3, nbr3, deg, wsn, wsr, bs, batch, fc1w, fc1b, fc3w, fc3b)


def _sc_softmax_body(src_hbm, dst_hbm, as_hbm, adm_hbm, z_hbm,
                     den_hbm, gidx_hbm, sidx_hbm,
                     srcb, dstb, asb, admb, exb, gib, sib, den_sh):
    c = lax.axis_index("c")
    s = lax.axis_index("s")
    rpt = NP // NTILE
    r0 = s * rpt
    pltpu.sync_copy(z_hbm.at[pl.ds(r0, rpt)], den_sh.at[pl.ds(r0, rpt)])
    plsc.subcore_barrier()
    epc = E // 2
    ept = epc // NTILE

    @pl.loop(0, ept // CH)
    def _(ci):
        base = c * epc + s * ept + ci * CH
        pltpu.sync_copy(src_hbm.at[pl.ds(base, CH)], srcb)
        pltpu.sync_copy(dst_hbm.at[pl.ds(base, CH)], dstb)
        pltpu.sync_copy(as_hbm.at[srcb], asb)
        pltpu.sync_copy(adm_hbm.at[dstb], admb)

        @plsc.parallel_loop(0, CH, unroll=4)
        def _(i):
            ei = asb[i, :] + admb[i, pl.ds(0, H)]
            e = jnp.maximum(ei, 0.2 * ei)
            exb[i, :] = jnp.exp(e - admb[i, pl.ds(H, H)])
            off = jnp.where(ei < 0.0, NP, 0).astype(jnp.int32)
            srcv = plsc.load_gather(srcb, [jnp.full((16,), i, jnp.int32)])
            dstv = plsc.load_gather(dstb, [jnp.full((16,), i, jnp.int32)])
            gib[i, :] = srcv + off
            sib[i, :] = dstv + off

        pltpu.sync_copy(exb, den_sh.at[dstb], add=True)
        pltpu.sync_copy(gib, gidx_hbm.at[pl.ds(base, CH)])
        pltpu.sync_copy(sib, sidx_hbm.at[pl.ds(base, CH)])

    plsc.subcore_barrier()

    @pl.when(c == 0)
    def _():
        pltpu.sync_copy(den_sh.at[pl.ds(r0, rpt)],
                        den_hbm.at[0, pl.ds(r0, rpt)])

    @pl.when(c == 1)
    def _():
        pltpu.sync_copy(den_sh.at[pl.ds(r0, rpt)],
                        den_hbm.at[1, pl.ds(r0, rpt)])


def _sc_softmax(src, dst, a_s, adm, z16):
    k = functools.partial(
        pl.kernel, mesh=_mesh, compiler_params=_sc_params,
        out_type=[jax.ShapeDtypeStruct((2, NP, H), jnp.float32),
                  jax.ShapeDtypeStruct((E, H), jnp.int32),
                  jax.ShapeDtypeStruct((E, H), jnp.int32)],
        scratch_types=[
            pltpu.VMEM((CH,), jnp.int32),
            pltpu.VMEM((CH,), jnp.int32),
            pltpu.VMEM((CH, H), jnp.float32),
            pltpu.VMEM((CH, 2 * H), jnp.float32),
            pltpu.VMEM((CH, H), jnp.float32),
            pltpu.VMEM((CH, H), jnp.int32),
            pltpu.VMEM((CH, H), jnp.int32),
            pltpu.VMEM_SHARED((NP, H), jnp.float32),
        ])(_sc_softmax_body)
    return k(src, dst, a_s, adm, z16)


def _make_stream_body(M, CHX, GRP, shared, cp):
    nch = (E // NTILE) // CHX
    ngroups = nch // GRP

    def body(t_hbm, g3_hbm, s3_hbm, z_hbm, out_hbm,
             gib, sib, rows, acc_sh, sg0, sg1, si0, si1):
        c = lax.axis_index("c")
        s = lax.axis_index("s")
        rpt = M // NTILE
        r0 = s * rpt
        q0 = s * nch

        sems = (sg0, sg1)

        @pl.loop(0, H // 2)
        def _(kl):
            k = 2 * kl + c
            kk = 0 if shared else k
            tref = t_hbm.at[k]

            isems = (si0, si1)

            def start_load_group(slot, gi):
                pltpu.async_copy(g3_hbm.at[kk, pl.ds(q0 + gi * GRP, GRP)],
                                 gib.at[slot], isems[slot])
                pltpu.async_copy(s3_hbm.at[kk, pl.ds(q0 + gi * GRP, GRP)],
                                 sib.at[slot], isems[slot])

            def wait_load_group(slot):
                pltpu.make_async_copy(g3_hbm.at[kk, pl.ds(q0, GRP)],
                                      gib.at[slot], isems[slot]).wait()
                pltpu.make_async_copy(s3_hbm.at[kk, pl.ds(q0, GRP)],
                                      sib.at[slot], isems[slot]).wait()

            def start_gather(buf, slot, row):
                pltpu.async_copy(tref.at[gib.at[slot, row]],
                                 rows.at[buf], sems[buf])

            def wait_gather(buf, slot):
                pltpu.make_async_copy(tref.at[gib.at[slot, 0]],
                                      rows.at[buf], sems[buf]).wait()

            def scatter(buf, slot, row):
                pltpu.sync_copy(rows.at[buf], acc_sh.at[sib.at[slot, row]],
                                add=True)

            pltpu.sync_copy(z_hbm.at[pl.ds(r0, rpt)],
                            acc_sh.at[pl.ds(r0, rpt)])
            plsc.subcore_barrier()

            start_load_group(0, 0)
            wait_load_group(0)
            start_gather(0, 0, 0)
            for g in range(ngroups):
                slot = g % 2
                if g + 1 < ngroups:
                    start_load_group(1 - slot, g + 1)

                @pl.loop(0, GRP // 2 - 1)
                def _(jj):
                    r = 2 * jj
                    start_gather(1, slot, r + 1)
                    wait_gather(0, slot)
                    scatter(0, slot, r)
                    start_gather(0, slot, r + 2)
                    wait_gather(1, slot)
                    scatter(1, slot, r + 1)

                r = GRP - 2
                start_gather(1, slot, r + 1)
                wait_gather(0, slot)
                scatter(0, slot, r)
                if g + 1 < ngroups:
                    wait_load_group(1 - slot)
                    start_gather(0, 1 - slot, 0)
                wait_gather(1, slot)
                scatter(1, slot, r + 1)

            plsc.subcore_barrier()
            pltpu.sync_copy(acc_sh.at[pl.ds(r0, rpt)],
                            out_hbm.at[k, pl.ds(r0, rpt)])

    return body


CHB2 = 200
CHC2 = 400
GRPB = 20
GRPC = 10


def _sc_gat_agg(t2, g3, s3, z2):
    cp = t2.shape[2]
    k = functools.partial(
        pl.kernel, mesh=_mesh, compiler_params=_sc_params,
        out_type=jax.ShapeDtypeStruct((H, 2 * NP, cp), jnp.float32),
        scratch_types=[
            pltpu.VMEM((2, GRPB, CHB2), jnp.int32),
            pltpu.VMEM((2, GRPB, CHB2), jnp.int32),
            pltpu.VMEM((2, CHB2, cp), jnp.float32),
            pltpu.VMEM_SHARED((2 * NP, cp), jnp.float32),
            pltpu.SemaphoreType.DMA,
            pltpu.SemaphoreType.DMA,
            pltpu.SemaphoreType.DMA,
            pltpu.SemaphoreType.DMA,
        ])(_make_stream_body(2 * NP, CHB2, GRPB, False, cp))
    return k(t2, g3, s3, z2)


def _sc_sum_agg(ht, src_e, dst_e, zc):
    cp = ht.shape[2]
    g3 = src_e.reshape(1, E // CHC2, CHC2)
    s3 = dst_e.reshape(1, E // CHC2, CHC2)
    k = functools.partial(
        pl.kernel, mesh=_mesh, compiler_params=_sc_params,
        out_type=jax.ShapeDtypeStruct((H, NP, cp), jnp.float32),
        scratch_types=[
            pltpu.VMEM((2, GRPC, CHC2), jnp.int32),
            pltpu.VMEM((2, GRPC, CHC2), jnp.int32),
            pltpu.VMEM((2, CHC2, cp), jnp.float32),
            pltpu.VMEM_SHARED((NP, cp), jnp.float32),
            pltpu.SemaphoreType.DMA,
            pltpu.SemaphoreType.DMA,
            pltpu.SemaphoreType.DMA,
            pltpu.SemaphoreType.DMA,
        ])(_make_stream_body(NP, CHC2, GRPC, True, cp))
    return k(ht, g3, s3, zc)


def _sc_sage_body(h3_hbm, src_hbm, dst_hbm, ones_hbm, z48_hbm, z16_hbm,
                  nbr_hbm, deg_hbm, srcb, dstb, rows, onesb,
                  nbr_sh, deg_sh):
    c = lax.axis_index("c")
    s = lax.axis_index("s")
    rpt = NP // NTILE
    r0 = s * rpt
    ept = E // NTILE

    @pl.when(c == 0)
    def _():
        pltpu.sync_copy(z48_hbm.at[pl.ds(r0, rpt)],
                        nbr_sh.at[pl.ds(r0, rpt)])
        plsc.subcore_barrier()

        @pl.loop(0, ept // CH)
        def _(ci):
            b = s * ept + ci * CH
            pltpu.sync_copy(src_hbm.at[pl.ds(b, CH)], srcb)
            pltpu.sync_copy(dst_hbm.at[pl.ds(b, CH)], dstb)
            pltpu.sync_copy(h3_hbm.at[srcb], rows)
            pltpu.sync_copy(rows, nbr_sh.at[dstb], add=True)

        plsc.subcore_barrier()
        pltpu.sync_copy(nbr_sh.at[pl.ds(r0, rpt)],
                        nbr_hbm.at[pl.ds(r0, rpt)])

    @pl.when(c == 1)
    def _():
        pltpu.sync_copy(z16_hbm.at[pl.ds(r0, rpt)],
                        deg_sh.at[pl.ds(r0, rpt)])
        pltpu.sync_copy(ones_hbm, onesb)
        plsc.subcore_barrier()

        @pl.loop(0, ept // CH)
        def _(ci):
            b = s * ept + ci * CH
            pltpu.sync_copy(dst_hbm.at[pl.ds(b, CH)], dstb)
            pltpu.sync_copy(onesb, deg_sh.at[dstb], add=True)

        plsc.subcore_barrier()
        pltpu.sync_copy(deg_sh.at[pl.ds(r0, rpt)],
                        deg_hbm.at[pl.ds(r0, rpt)])


def _sc_sage(h3, src, dst, ones, z48, z16):
    k = functools.partial(
        pl.kernel, mesh=_mesh, compiler_params=_sc_params,
        out_type=[jax.ShapeDtypeStruct((NP, 48), jnp.float32),
                  jax.ShapeDtypeStruct((NP, H), jnp.float32)],
        scratch_types=[
            pltpu.VMEM((CH,), jnp.int32),
            pltpu.VMEM((CH,), jnp.int32),
            pltpu.VMEM((CH, 48), jnp.float32),
            pltpu.VMEM((CH, H), jnp.float32),
            pltpu.VMEM_SHARED((N, 48), jnp.float32),
            pltpu.VMEM_SHARED((NP, H), jnp.float32),
        ])(_sc_sage_body)
    return k(h3, src, dst, ones, z48, z16)


def _gat_layer(x, src, dst, wt, asr, adr, bp, z16, z2):
    ht = _mm(x, wt)
    a_s, a_d = _asad(ht, asr, adr)
    adm, exself = _prep(a_s, a_d)
    den, gidx, sidx = _sc_softmax(src, dst, a_s, adm, z16)
    gidx_t = _tr(gidx)
    sidx_t = _tr(sidx)
    t2 = _scale_tables(ht, a_s)
    acc = _sc_gat_agg(t2, gidx_t, sidx_t, z2).reshape(H, 2, NP, ht.shape[2])
    return _combine(acc, ht, exself, den, adm, bp)


def kernel(x, edge_index, batch, W1, att_src1, att_dst1, b1, W2, att_src2,
           att_dst2, b2, Wg_root, Wg_nbr, bg, Ws_nbr, Ws_root, bs, fc1_w,
           fc1_b, fc3_w, fc3_b):
    f32 = jnp.float32
    src = edge_index[0]
    dst = edge_index[1]
    C1 = att_src1.shape[1]
    C2 = att_src2.shape[1]

    def padh(a, c, cp):
        return jnp.pad(a.reshape(H, 1, c), ((0, 0), (0, 0), (0, cp - c)))

    w1t = jnp.pad(W1.reshape(-1, H, C1), ((0, 0), (0, 0), (0, CP - C1))
                  ).transpose(1, 0, 2)
    w2t = jnp.pad(W2.reshape(H, C1, H, C2),
                  ((0, 0), (0, CP - C1), (0, 0), (0, CP2 - C2))
                  ).reshape(H * CP, H, CP2).transpose(1, 0, 2)
    wgr = jnp.pad(Wg_root.reshape(H, C2, 40),
                  ((0, 0), (0, CP2 - C2), (0, 8))).reshape(H * CP2, 48)
    wgn = jnp.pad(Wg_nbr.reshape(H, C2, 40),
                  ((0, 0), (0, CP2 - C2), (0, 8)))
    wsn = jnp.pad(Ws_nbr, ((0, 8), (0, 8)))
    wsr = jnp.pad(Ws_root, ((0, 8), (0, 8)))
    fc1p = jnp.pad(fc1_w, ((0, 8), (0, 0)))

    z16 = jnp.zeros((NP, H), f32)
    z2a = jnp.zeros((2 * NP, CP), f32)
    z2b = jnp.zeros((2 * NP, CP2), f32)
    z48 = jnp.zeros((NP, 48), f32)
    ones = jnp.ones((CH, H), f32)

    x = jnp.pad(x, ((0, NP - N), (0, 0)))
    batch = jnp.pad(batch, (0, NP - N), constant_values=G)

    h2nm, _ = _gat_layer(x, src, dst, w1t, padh(att_src1, C1, CP),
                         padh(att_dst1, C1, CP), padh(b1, C1, CP), z16, z2a)
    h2 = h2nm
    h3nm, h3hm = _gat_layer(h2, src, dst, w2t, padh(att_src2, C2, CP2),
                            padh(att_dst2, C2, CP2), padh(b2, C2, CP2),
                            z16, z2b)
    h3n = h3nm

    nbrg = _sc_sum_agg(h3hm, src, dst, z48)
    h4 = _graphconv(h3n, nbrg, wgr, wgn, jnp.pad(bg, (0, 8)).reshape(1, 48))

    nbr3, deg = _sc_sage(h4, src, dst, ones, z48, z16)
    out = _head(h4, nbr3, deg[:, :1], wsn, wsr,
                jnp.pad(bs, (0, 8)).reshape(1, 48), batch.reshape(NP, 1),
                fc1p, fc1_b.reshape(1, 32), fc3_w, fc3_b.reshape(1, 10))
    return out
